# Initial kernel scaffold; baseline (speedup 1.0000x reference)
import jax
import jax.numpy as jnp
from jax.experimental import pallas as pl


def kernel(ref_bxyz, query_bxyz, ref_feat, query_feat, e_ref, e_query, W_f0, gamma_f0, beta_f0, W_s0, gamma_s0, beta_s0, W1, b1, gamma1, beta1):
    raise NotImplementedError("write your pallas kernel here")


# speedup vs baseline: 17.4298x; 17.4298x over previous
import functools

import jax
import jax.numpy as jnp
from jax import lax
from jax.experimental import pallas as pl
from jax.experimental.pallas import tpu as pltpu
from jax.experimental.pallas import tpu_sc as plsc

N_REF = 10000
N_Q = 100000
E = 300000
D = 128

NC, NS = 2, 16
NW = NC * NS
RW = 3136
NQ_PAD = NW * RW
WIN = 512
KB = 128
EBLK = E // 16


def _rsqrt_nr(s):
    i = lax.bitcast_convert_type(s, jnp.int32)
    i = jnp.int32(0x5F3759DF) - (i >> 1)
    r = lax.bitcast_convert_type(i, jnp.float32)
    for _ in range(3):
        r = r * (1.5 - 0.5 * s * r * r)
    return r


def _sc_body(rx_h, ry_h, rz_h, qx_h, qy_h, qz_h, er_h, eq_h, feat_h, zero_h,
             qf_h, ws_h,
             rx_v, ry_v, rz_v, qx_v, qy_v, qz_v,
             acc_v, wsw_v, feat_v, er_v, eq_v, d_v, bs_v,
             sem_g, sem_f, sem_w, sem_b):
    wid = lax.axis_index("c") * NS + lax.axis_index("s")
    row_lo = wid * RW
    row_hi = jnp.minimum(row_lo + RW, N_Q)

    def searchsorted(v):
        def body(_, c):
            lo, hi = c
            mid = (lo + hi) >> 1
            pltpu.async_copy(eq_h.at[pl.ds(pl.multiple_of(mid * 16, 16), 16)],
                             bs_v, sem_b).wait()
            below = bs_v[...][0] < v
            return jnp.where(below, mid, lo), jnp.where(below, hi, mid)

        lo, _ = lax.fori_loop(0, 15, body, (jnp.int32(0), jnp.int32(EBLK)))
        pltpu.async_copy(eq_h.at[pl.ds(pl.multiple_of(lo * 16, 16), 16)],
                         bs_v, sem_b).wait()
        cnt = jnp.sum((bs_v[...] < v).astype(jnp.int32))
        return lo * 16 + cnt

    e_lo = searchsorted(row_lo)
    e_hi = searchsorted(row_hi)
    a_lo = e_lo & jnp.int32(-8)

    row_lo8 = pl.multiple_of(row_lo, 64)
    pltpu.async_copy(rx_h, rx_v, sem_b)
    pltpu.async_copy(ry_h, ry_v, sem_b)
    pltpu.async_copy(rz_h, rz_v, sem_b)
    pltpu.async_copy(qx_h.at[pl.ds(row_lo8, RW)], qx_v, sem_b)
    pltpu.async_copy(qy_h.at[pl.ds(row_lo8, RW)], qy_v, sem_b)
    pltpu.async_copy(qz_h.at[pl.ds(row_lo8, RW)], qz_v, sem_b)
    pltpu.async_copy(zero_h, acc_v, sem_b)
    pltpu.async_copy(zero_h.at[pl.ds(0, WIN)], wsw_v, sem_b)
    pltpu.make_async_copy(rx_h, rx_v, sem_b).wait()
    pltpu.make_async_copy(ry_h, ry_v, sem_b).wait()
    pltpu.make_async_copy(rz_h, rz_v, sem_b).wait()
    pltpu.make_async_copy(qx_h.at[pl.ds(row_lo8, RW)], qx_v, sem_b).wait()
    pltpu.make_async_copy(qy_h.at[pl.ds(row_lo8, RW)], qy_v, sem_b).wait()
    pltpu.make_async_copy(qz_h.at[pl.ds(row_lo8, RW)], qz_v, sem_b).wait()
    pltpu.make_async_copy(zero_h, acc_v, sem_b).wait()
    pltpu.make_async_copy(zero_h.at[pl.ds(0, WIN)], wsw_v, sem_b).wait()

    def flush(wb, nrows):
        n32 = nrows >> 5

        def issue(t, _):
            o1 = pl.multiple_of((wb + t * 32) * D, 4096)
            o2 = pl.multiple_of(wb + t * 32, 32)
            pltpu.async_copy(acc_v.at[pl.ds(t * 4096, 4096)],
                             qf_h.at[pl.ds(o1, 4096)], sem_f)
            pltpu.async_copy(wsw_v.at[pl.ds(t * 32, 32)],
                             ws_h.at[pl.ds(o2, 32)], sem_w)
            return 0

        def drain(t, _):
            o1 = pl.multiple_of((wb + t * 32) * D, 4096)
            o2 = pl.multiple_of(wb + t * 32, 32)
            pltpu.make_async_copy(acc_v.at[pl.ds(t * 4096, 4096)],
                                  qf_h.at[pl.ds(o1, 4096)],
                                  sem_f).wait()
            pltpu.make_async_copy(wsw_v.at[pl.ds(t * 32, 32)],
                                  ws_h.at[pl.ds(o2, 32)],
                                  sem_w).wait()
            return 0

        lax.fori_loop(0, n32, issue, 0)
        lax.fori_loop(0, n32, drain, 0)
        pltpu.async_copy(zero_h, acc_v, sem_b)
        pltpu.async_copy(zero_h.at[pl.ds(0, WIN)], wsw_v, sem_b)
        pltpu.make_async_copy(zero_h, acc_v, sem_b).wait()
        pltpu.make_async_copy(zero_h.at[pl.ds(0, WIN)], wsw_v, sem_b).wait()

    def advance(wb):
        flush(wb, jnp.minimum(WIN, row_hi - wb))
        return wb + WIN

    lane0 = lax.iota(jnp.int32, 16) == 0

    def edge_body(i, wb):
        iv = jnp.full((16,), i, jnp.int32)
        q = plsc.load_gather(eq_v, [iv])[0]
        db = plsc.load_gather(d_v, [iv])

        def do(wb):
            wb = lax.fori_loop(0, (q - wb) >> 9, lambda _, w: advance(w), wb)
            ro = q - wb
            for j in range(D // 16):
                f = feat_v[i, pl.ds(16 * j, 16)]
                plsc.addupdate(acc_v.at[pl.ds(ro * D + 16 * j, 16)], f * db)
            plsc.addupdate_scatter(wsw_v, [jnp.full((16,), ro, jnp.int32)],
                                   db, mask=lane0)
            return wb

        return lax.cond(q >= row_lo, do, lambda w: w, wb)

    def block_body(b, wb):
        base = pl.multiple_of(a_lo + b * KB, 8)
        n = jnp.minimum(e_hi - base, KB)
        pltpu.async_copy(er_h.at[pl.ds(base, KB)], er_v, sem_b)
        pltpu.async_copy(eq_h.at[pl.ds(base, KB)], eq_v, sem_b)
        pltpu.make_async_copy(er_h.at[pl.ds(base, KB)], er_v, sem_b).wait()
        pltpu.make_async_copy(eq_h.at[pl.ds(base, KB)], eq_v, sem_b).wait()
        gat = pltpu.async_copy(feat_h.at[er_v], feat_v, sem_g)
        for g in range(KB // 16):
            qi = eq_v[pl.ds(16 * g, 16)]
            ri = er_v[pl.ds(16 * g, 16)]
            valid = (qi >= row_lo) & (qi < row_hi)
            qidx = jnp.where(valid, qi - row_lo, 0)
            qx = plsc.load_gather(qx_v, [qidx], mask=valid)
            qy = plsc.load_gather(qy_v, [qidx], mask=valid)
            qz = plsc.load_gather(qz_v, [qidx], mask=valid)
            rxx = plsc.load_gather(rx_v, [ri])
            ryy = plsc.load_gather(ry_v, [ri])
            rzz = plsc.load_gather(rz_v, [ri])
            dx = rxx - qx
            dy = ryy - qy
            dz = rzz - qz
            s = jnp.maximum(dx * dx + dy * dy + dz * dz, 1e-30)
            r = _rsqrt_nr(s)
            dd = r / (1.0 + 1e-8 * r)
            d_v[pl.ds(16 * g, 16)] = jnp.where(valid, dd, 0.0)
        gat.wait()
        return lax.fori_loop(0, n, edge_body, wb)

    nblk = (e_hi - a_lo + (KB - 1)) >> 7
    wb = lax.fori_loop(0, nblk, block_body, row_lo)
    lax.fori_loop(0, (row_hi - wb + (WIN - 1)) >> 9, lambda _, w: advance(w), wb)


def _sc_edge_conv(ref_bxyz, query_bxyz, e_ref, e_query, feat):
    mesh = plsc.VectorSubcoreMesh(core_axis_name="c", subcore_axis_name="s")
    rx = ref_bxyz[:, 1] + 0.0
    ry = ref_bxyz[:, 2] + 0.0
    rz = ref_bxyz[:, 3] + 0.0
    qpad = jnp.pad(query_bxyz[:, 1:4], ((0, NQ_PAD - N_Q), (0, 0)))
    qx = qpad[:, 0] + 0.0
    qy = qpad[:, 1] + 0.0
    qz = qpad[:, 2] + 0.0
    er_p = jnp.pad(e_ref, (0, KB))
    eq_p = jnp.pad(e_query, (0, KB))
    zeros = jnp.zeros((WIN * D,), jnp.float32)
    run = pl.kernel(
        _sc_body,
        out_type=(jax.ShapeDtypeStruct((N_Q * D,), jnp.float32),
                  jax.ShapeDtypeStruct((N_Q,), jnp.float32)),
        mesh=mesh,
        compiler_params=pltpu.CompilerParams(needs_layout_passes=False),
        scratch_types=[
            pltpu.VMEM((N_REF,), jnp.float32),
            pltpu.VMEM((N_REF,), jnp.float32),
            pltpu.VMEM((N_REF,), jnp.float32),
            pltpu.VMEM((RW,), jnp.float32),
            pltpu.VMEM((RW,), jnp.float32),
            pltpu.VMEM((RW,), jnp.float32),
            pltpu.VMEM((WIN * D,), jnp.float32),
            pltpu.VMEM((WIN,), jnp.float32),
            pltpu.VMEM((KB, D), jnp.float32),
            pltpu.VMEM((KB,), jnp.int32),
            pltpu.VMEM((KB,), jnp.int32),
            pltpu.VMEM((KB,), jnp.float32),
            pltpu.VMEM((16,), jnp.int32),
            pltpu.SemaphoreType.DMA,
            pltpu.SemaphoreType.DMA,
            pltpu.SemaphoreType.DMA,
            pltpu.SemaphoreType.DMA,
        ],
    )
    qf_flat, wsum = run(rx, ry, rz, qx, qy, qz, er_p, eq_p, feat, zeros)
    return qf_flat.reshape(N_Q, D), wsum


def _k1_body(x_ref, wt_ref, g_ref, b_ref, o_ref):
    y = jnp.dot(x_ref[...], wt_ref[...], preferred_element_type=jnp.float32)
    m = jnp.mean(y, axis=0, keepdims=True)
    v = jnp.mean((y - m) ** 2, axis=0, keepdims=True)
    o_ref[...] = g_ref[...] * (y - m) / jnp.sqrt(v + 1e-5) + b_ref[...]


def _ref_branch(ref_feat, W_f0, gamma, beta):
    return pl.pallas_call(
        _k1_body,
        out_shape=jax.ShapeDtypeStruct((N_REF, D), jnp.float32),
    )(ref_feat, W_f0.T, gamma.reshape(1, D), beta.reshape(1, D))


_BQ = 1000
_NBQ = N_Q // _BQ


def _stats_body(x_ref, wt_ref, ssum_ref, ssq_ref, acc_ref):
    i = pl.program_id(0)

    @pl.when(i == 0)
    def _():
        acc_ref[...] = jnp.zeros_like(acc_ref)

    y = jnp.dot(x_ref[...], wt_ref[...], preferred_element_type=jnp.float32)
    acc_ref[0:1, :] += jnp.sum(y, axis=0, keepdims=True)
    acc_ref[1:2, :] += jnp.sum(y * y, axis=0, keepdims=True)

    @pl.when(i == _NBQ - 1)
    def _():
        ssum_ref[...] = acc_ref[0:1, :]
        ssq_ref[...] = acc_ref[1:2, :]


def _skip_stats(query_feat, W_s0):
    return pl.pallas_call(
        _stats_body,
        grid=(_NBQ,),
        in_specs=[
            pl.BlockSpec((_BQ, D), lambda i: (i, 0)),
            pl.BlockSpec((D, D), lambda i: (0, 0)),
        ],
        out_specs=[
            pl.BlockSpec((1, D), lambda i: (0, 0)),
            pl.BlockSpec((1, D), lambda i: (0, 0)),
        ],
        out_shape=[
            jax.ShapeDtypeStruct((1, D), jnp.float32),
            jax.ShapeDtypeStruct((1, D), jnp.float32),
        ],
        scratch_shapes=[pltpu.VMEM((2, D), jnp.float32)],
    )(query_feat, W_s0.T)


def _mid_body(x_ref, wt_ref, sc_ref, bi_ref, acc_ref, w_ref, w1t_ref, b1_ref,
              z_ref, zsum_ref, zsq_ref, st_ref):
    i = pl.program_id(0)

    @pl.when(i == 0)
    def _():
        st_ref[...] = jnp.zeros_like(st_ref)

    ys = jnp.dot(x_ref[...], wt_ref[...], preferred_element_type=jnp.float32)
    skip = ys * sc_ref[...] + bi_ref[...]
    w = w_ref[...].reshape(_BQ, 1)
    inv = jnp.where(w > 0, 1.0 / w, 0.0)
    h = jnp.maximum(acc_ref[...] * inv + skip, 0.0)
    z = jnp.dot(h, w1t_ref[...], preferred_element_type=jnp.float32) + b1_ref[...]
    z_ref[...] = z
    st_ref[0:1, :] += jnp.sum(z, axis=0, keepdims=True)
    st_ref[1:2, :] += jnp.sum(z * z, axis=0, keepdims=True)

    @pl.when(i == _NBQ - 1)
    def _():
        zsum_ref[...] = st_ref[0:1, :]
        zsq_ref[...] = st_ref[1:2, :]


def _mid(query_feat, W_s0, scale_s, bias_s, qf_acc, wsum, W1, b1):
    return pl.pallas_call(
        _mid_body,
        grid=(_NBQ,),
        in_specs=[
            pl.BlockSpec((_BQ, D), lambda i: (i, 0)),
            pl.BlockSpec((D, D), lambda i: (0, 0)),
            pl.BlockSpec((1, D), lambda i: (0, 0)),
            pl.BlockSpec((1, D), lambda i: (0, 0)),
            pl.BlockSpec((_BQ, D), lambda i: (i, 0)),
            pl.BlockSpec((1, 1, _BQ), lambda i: (i, 0, 0)),
            pl.BlockSpec((D, D), lambda i: (0, 0)),
            pl.BlockSpec((1, D), lambda i: (0, 0)),
        ],
        out_specs=[
            pl.BlockSpec((_BQ, D), lambda i: (i, 0)),
            pl.BlockSpec((1, D), lambda i: (0, 0)),
            pl.BlockSpec((1, D), lambda i: (0, 0)),
        ],
        out_shape=[
            jax.ShapeDtypeStruct((N_Q, D), jnp.float32),
            jax.ShapeDtypeStruct((1, D), jnp.float32),
            jax.ShapeDtypeStruct((1, D), jnp.float32),
        ],
        scratch_shapes=[pltpu.VMEM((2, D), jnp.float32)],
    )(query_feat, W_s0.T, scale_s, bias_s, qf_acc,
      wsum.reshape(_NBQ, 1, _BQ), W1.T, b1.reshape(1, D))


def _fin_body(z_ref, sc_ref, bi_ref, o_ref):
    o_ref[...] = jnp.maximum(z_ref[...] * sc_ref[...] + bi_ref[...], 0.0)


def _final(z, scale_z, bias_z):
    return pl.pallas_call(
        _fin_body,
        grid=(_NBQ,),
        in_specs=[
            pl.BlockSpec((_BQ, D), lambda i: (i, 0)),
            pl.BlockSpec((1, D), lambda i: (0, 0)),
            pl.BlockSpec((1, D), lambda i: (0, 0)),
        ],
        out_specs=pl.BlockSpec((_BQ, D), lambda i: (i, 0)),
        out_shape=jax.ShapeDtypeStruct((N_Q, D), jnp.float32),
    )(z, scale_z, bias_z)


def kernel(ref_bxyz, query_bxyz, ref_feat, query_feat, e_ref, e_query,
           W_f0, gamma_f0, beta_f0, W_s0, gamma_s0, beta_s0,
           W1, b1, gamma1, beta1):
    feat2 = _ref_branch(ref_feat, W_f0, gamma_f0, beta_f0)
    qf_acc, wsum = _sc_edge_conv(ref_bxyz, query_bxyz, e_ref, e_query, feat2)
    ssum, ssq = _skip_stats(query_feat, W_s0)
    n = jnp.float32(N_Q)
    m_s = ssum / n
    v_s = ssq / n - m_s * m_s
    scale_s = gamma_s0.reshape(1, D) / jnp.sqrt(v_s + 1e-5)
    bias_s = beta_s0.reshape(1, D) - m_s * scale_s
    z, zsum, zsq = _mid(query_feat, W_s0, scale_s, bias_s, qf_acc, wsum, W1, b1)
    m_z = zsum / n
    v_z = zsq / n - m_z * m_z
    scale_z = gamma1.reshape(1, D) / jnp.sqrt(v_z + 1e-5)
    bias_z = beta1.reshape(1, D) - m_z * scale_z
    return _final(z, scale_z, bias_z)


# speedup vs baseline: 22.7412x; 1.3047x over previous
import functools

import jax
import jax.numpy as jnp
from jax import lax
from jax.experimental import pallas as pl
from jax.experimental.pallas import tpu as pltpu
from jax.experimental.pallas import tpu_sc as plsc

N_REF = 10000
N_Q = 100000
E = 300000
D = 128

NC, NS = 2, 16
NW = NC * NS
RW = 3136
NQ_PAD = NW * RW
WIN = 512
KB = 128
EBLK = E // 16


def _rsqrt_nr(s):
    i = lax.bitcast_convert_type(s, jnp.int32)
    i = jnp.int32(0x5F3759DF) - (i >> 1)
    r = lax.bitcast_convert_type(i, jnp.float32)
    for _ in range(3):
        r = r * (1.5 - 0.5 * s * r * r)
    return r


def _sc_body(rx_h, ry_h, rz_h, qx_h, qy_h, qz_h, er_h, eq_h, feat_h, zero_h,
             qf_h, ws_h,
             rx_v, ry_v, rz_v, qx_v, qy_v, qz_v,
             acc_v, wsw_v, feat_v, er_v, eq_v, d_v, qc_v, bs_v,
             sem_g, sem_f, sem_w, sem_b):
    wid = lax.axis_index("c") * NS + lax.axis_index("s")
    row_lo = wid * RW
    row_hi = jnp.minimum(row_lo + RW, N_Q)

    def searchsorted(v):
        def body(_, c):
            lo, hi = c
            mid = (lo + hi) >> 1
            pltpu.async_copy(eq_h.at[pl.ds(pl.multiple_of(mid * 16, 16), 16)],
                             bs_v, sem_b).wait()
            below = bs_v[...][0] < v
            return jnp.where(below, mid, lo), jnp.where(below, hi, mid)

        lo, _ = lax.fori_loop(0, 15, body, (jnp.int32(0), jnp.int32(EBLK)))
        pltpu.async_copy(eq_h.at[pl.ds(pl.multiple_of(lo * 16, 16), 16)],
                         bs_v, sem_b).wait()
        cnt = jnp.sum((bs_v[...] < v).astype(jnp.int32))
        return lo * 16 + cnt

    e_lo = searchsorted(row_lo)
    e_hi = searchsorted(row_hi)
    a_lo = e_lo & jnp.int32(-8)

    row_lo8 = pl.multiple_of(row_lo, 64)
    pltpu.async_copy(rx_h, rx_v, sem_b)
    pltpu.async_copy(ry_h, ry_v, sem_b)
    pltpu.async_copy(rz_h, rz_v, sem_b)
    pltpu.async_copy(qx_h.at[pl.ds(row_lo8, RW)], qx_v, sem_b)
    pltpu.async_copy(qy_h.at[pl.ds(row_lo8, RW)], qy_v, sem_b)
    pltpu.async_copy(qz_h.at[pl.ds(row_lo8, RW)], qz_v, sem_b)
    pltpu.async_copy(zero_h, acc_v, sem_b)
    pltpu.async_copy(zero_h.at[pl.ds(0, WIN)], wsw_v, sem_b)
    pltpu.make_async_copy(rx_h, rx_v, sem_b).wait()
    pltpu.make_async_copy(ry_h, ry_v, sem_b).wait()
    pltpu.make_async_copy(rz_h, rz_v, sem_b).wait()
    pltpu.make_async_copy(qx_h.at[pl.ds(row_lo8, RW)], qx_v, sem_b).wait()
    pltpu.make_async_copy(qy_h.at[pl.ds(row_lo8, RW)], qy_v, sem_b).wait()
    pltpu.make_async_copy(qz_h.at[pl.ds(row_lo8, RW)], qz_v, sem_b).wait()
    pltpu.make_async_copy(zero_h, acc_v, sem_b).wait()
    pltpu.make_async_copy(zero_h.at[pl.ds(0, WIN)], wsw_v, sem_b).wait()

    def flush(wb, nrows):
        n32 = nrows >> 5

        def issue(t, _):
            o1 = pl.multiple_of((wb + t * 32) * D, 4096)
            o2 = pl.multiple_of(wb + t * 32, 32)
            pltpu.async_copy(acc_v.at[pl.ds(t * 4096, 4096)],
                             qf_h.at[pl.ds(o1, 4096)], sem_f)
            pltpu.async_copy(wsw_v.at[pl.ds(t * 32, 32)],
                             ws_h.at[pl.ds(o2, 32)], sem_w)
            return 0

        def drain(t, _):
            o1 = pl.multiple_of((wb + t * 32) * D, 4096)
            o2 = pl.multiple_of(wb + t * 32, 32)
            pltpu.make_async_copy(acc_v.at[pl.ds(t * 4096, 4096)],
                                  qf_h.at[pl.ds(o1, 4096)],
                                  sem_f).wait()
            pltpu.make_async_copy(wsw_v.at[pl.ds(t * 32, 32)],
                                  ws_h.at[pl.ds(o2, 32)],
                                  sem_w).wait()
            return 0

        lax.fori_loop(0, n32, issue, 0)
        lax.fori_loop(0, n32, drain, 0)
        pltpu.async_copy(zero_h, acc_v, sem_b)
        pltpu.async_copy(zero_h.at[pl.ds(0, WIN)], wsw_v, sem_b)
        pltpu.make_async_copy(zero_h, acc_v, sem_b).wait()
        pltpu.make_async_copy(zero_h.at[pl.ds(0, WIN)], wsw_v, sem_b).wait()

    def advance(wb):
        flush(wb, jnp.minimum(WIN, row_hi - wb))
        return wb + WIN

    lane = lax.iota(jnp.int32, 16)
    lane0 = lane == 0

    def group_body(g, wb):
        eqb = qc_v[pl.ds(16 * g, 16)]
        db = d_v[pl.ds(16 * g, 16)]
        q0 = eqb[0]
        q15 = eqb[15]
        wb = lax.fori_loop(0, (q0 - wb) >> 9, lambda _, w: advance(w), wb)

        def fast(wb):
            for k in range(16):
                ro = eqb[k] - wb
                dv = jnp.full((16,), db[k])
                for j in range(D // 16):
                    f = feat_v[16 * g + k, pl.ds(16 * j, 16)]
                    plsc.addupdate(acc_v.at[pl.ds(ro * D + 16 * j, 16)],
                                   f * dv)
                plsc.addupdate_scatter(wsw_v,
                                       [jnp.full((16,), ro, jnp.int32)],
                                       db, mask=lane == k)
            return wb

        def slow(wb):
            def eb(k, w):
                iv = jnp.full((16,), 16 * g + k, jnp.int32)
                q = plsc.load_gather(qc_v, [iv])[0]
                dbv = plsc.load_gather(d_v, [iv])
                w = lax.fori_loop(0, (q - w) >> 9, lambda _, x: advance(x), w)
                ro = q - w
                for j in range(D // 16):
                    f = feat_v[16 * g + k, pl.ds(16 * j, 16)]
                    plsc.addupdate(acc_v.at[pl.ds(ro * D + 16 * j, 16)],
                                   f * dbv)
                plsc.addupdate_scatter(wsw_v,
                                       [jnp.full((16,), ro, jnp.int32)],
                                       dbv, mask=lane0)
                return w

            return lax.fori_loop(0, 16, eb, wb)

        return lax.cond(q15 - wb < WIN, fast, slow, wb)

    def block_body(b, wb):
        base = pl.multiple_of(a_lo + b * KB, 8)
        pltpu.async_copy(er_h.at[pl.ds(base, KB)], er_v, sem_b)
        pltpu.async_copy(eq_h.at[pl.ds(base, KB)], eq_v, sem_b)
        pltpu.make_async_copy(er_h.at[pl.ds(base, KB)], er_v, sem_b).wait()
        pltpu.make_async_copy(eq_h.at[pl.ds(base, KB)], eq_v, sem_b).wait()
        gat = pltpu.async_copy(feat_h.at[er_v], feat_v, sem_g)
        for g in range(KB // 16):
            qi = eq_v[pl.ds(16 * g, 16)]
            ri = er_v[pl.ds(16 * g, 16)]
            valid = (qi >= row_lo) & (qi < row_hi)
            qidx = jnp.where(valid, qi - row_lo, 0)
            qx = plsc.load_gather(qx_v, [qidx], mask=valid)
            qy = plsc.load_gather(qy_v, [qidx], mask=valid)
            qz = plsc.load_gather(qz_v, [qidx], mask=valid)
            rxx = plsc.load_gather(rx_v, [ri])
            ryy = plsc.load_gather(ry_v, [ri])
            rzz = plsc.load_gather(rz_v, [ri])
            dx = rxx - qx
            dy = ryy - qy
            dz = rzz - qz
            s = jnp.maximum(dx * dx + dy * dy + dz * dz, 1e-30)
            r = _rsqrt_nr(s)
            dd = r / (1.0 + 1e-8 * r)
            d_v[pl.ds(16 * g, 16)] = jnp.where(valid, dd, 0.0)
            qc_v[pl.ds(16 * g, 16)] = jnp.minimum(
                jnp.maximum(qi, row_lo), row_hi - 1)
        gat.wait()
        return lax.fori_loop(0, KB // 16, group_body, wb)

    nblk = (e_hi - a_lo + (KB - 1)) >> 7
    wb = lax.fori_loop(0, nblk, block_body, row_lo)
    lax.fori_loop(0, (row_hi - wb + (WIN - 1)) >> 9, lambda _, w: advance(w), wb)


def _sc_edge_conv(ref_bxyz, query_bxyz, e_ref, e_query, feat):
    mesh = plsc.VectorSubcoreMesh(core_axis_name="c", subcore_axis_name="s")
    rx = ref_bxyz[:, 1] + 0.0
    ry = ref_bxyz[:, 2] + 0.0
    rz = ref_bxyz[:, 3] + 0.0
    qpad = jnp.pad(query_bxyz[:, 1:4], ((0, NQ_PAD - N_Q), (0, 0)))
    qx = qpad[:, 0] + 0.0
    qy = qpad[:, 1] + 0.0
    qz = qpad[:, 2] + 0.0
    er_p = jnp.pad(e_ref, (0, KB))
    eq_p = jnp.pad(e_query, (0, KB), constant_values=N_Q)
    zeros = jnp.zeros((WIN * D,), jnp.float32)
    run = pl.kernel(
        _sc_body,
        out_type=(jax.ShapeDtypeStruct((N_Q * D,), jnp.float32),
                  jax.ShapeDtypeStruct((N_Q,), jnp.float32)),
        mesh=mesh,
        compiler_params=pltpu.CompilerParams(needs_layout_passes=False),
        scratch_types=[
            pltpu.VMEM((N_REF,), jnp.float32),
            pltpu.VMEM((N_REF,), jnp.float32),
            pltpu.VMEM((N_REF,), jnp.float32),
            pltpu.VMEM((RW,), jnp.float32),
            pltpu.VMEM((RW,), jnp.float32),
            pltpu.VMEM((RW,), jnp.float32),
            pltpu.VMEM((WIN * D,), jnp.float32),
            pltpu.VMEM((WIN,), jnp.float32),
            pltpu.VMEM((KB, D), jnp.float32),
            pltpu.VMEM((KB,), jnp.int32),
            pltpu.VMEM((KB,), jnp.int32),
            pltpu.VMEM((KB,), jnp.float32),
            pltpu.VMEM((KB,), jnp.int32),
            pltpu.VMEM((16,), jnp.int32),
            pltpu.SemaphoreType.DMA,
            pltpu.SemaphoreType.DMA,
            pltpu.SemaphoreType.DMA,
            pltpu.SemaphoreType.DMA,
        ],
    )
    qf_flat, wsum = run(rx, ry, rz, qx, qy, qz, er_p, eq_p, feat, zeros)
    return qf_flat.reshape(N_Q, D), wsum


def _k1_body(x_ref, wt_ref, g_ref, b_ref, o_ref):
    y = jnp.dot(x_ref[...], wt_ref[...], preferred_element_type=jnp.float32)
    m = jnp.mean(y, axis=0, keepdims=True)
    v = jnp.mean((y - m) ** 2, axis=0, keepdims=True)
    o_ref[...] = g_ref[...] * (y - m) / jnp.sqrt(v + 1e-5) + b_ref[...]


def _ref_branch(ref_feat, W_f0, gamma, beta):
    return pl.pallas_call(
        _k1_body,
        out_shape=jax.ShapeDtypeStruct((N_REF, D), jnp.float32),
    )(ref_feat, W_f0.T, gamma.reshape(1, D), beta.reshape(1, D))


_BQ = 1000
_NBQ = N_Q // _BQ


def _stats_body(x_ref, wt_ref, ssum_ref, ssq_ref, acc_ref):
    i = pl.program_id(0)

    @pl.when(i == 0)
    def _():
        acc_ref[...] = jnp.zeros_like(acc_ref)

    y = jnp.dot(x_ref[...], wt_ref[...], preferred_element_type=jnp.float32)
    acc_ref[0:1, :] += jnp.sum(y, axis=0, keepdims=True)
    acc_ref[1:2, :] += jnp.sum(y * y, axis=0, keepdims=True)

    @pl.when(i == _NBQ - 1)
    def _():
        ssum_ref[...] = acc_ref[0:1, :]
        ssq_ref[...] = acc_ref[1:2, :]


def _skip_stats(query_feat, W_s0):
    return pl.pallas_call(
        _stats_body,
        grid=(_NBQ,),
        in_specs=[
            pl.BlockSpec((_BQ, D), lambda i: (i, 0)),
            pl.BlockSpec((D, D), lambda i: (0, 0)),
        ],
        out_specs=[
            pl.BlockSpec((1, D), lambda i: (0, 0)),
            pl.BlockSpec((1, D), lambda i: (0, 0)),
        ],
        out_shape=[
            jax.ShapeDtypeStruct((1, D), jnp.float32),
            jax.ShapeDtypeStruct((1, D), jnp.float32),
        ],
        scratch_shapes=[pltpu.VMEM((2, D), jnp.float32)],
    )(query_feat, W_s0.T)


def _mid_body(x_ref, wt_ref, sc_ref, bi_ref, acc_ref, w_ref, w1t_ref, b1_ref,
              z_ref, zsum_ref, zsq_ref, st_ref):
    i = pl.program_id(0)

    @pl.when(i == 0)
    def _():
        st_ref[...] = jnp.zeros_like(st_ref)

    ys = jnp.dot(x_ref[...], wt_ref[...], preferred_element_type=jnp.float32)
    skip = ys * sc_ref[...] + bi_ref[...]
    w = w_ref[...].reshape(_BQ, 1)
    inv = jnp.where(w > 0, 1.0 / w, 0.0)
    h = jnp.maximum(acc_ref[...] * inv + skip, 0.0)
    z = jnp.dot(h, w1t_ref[...], preferred_element_type=jnp.float32) + b1_ref[...]
    z_ref[...] = z
    st_ref[0:1, :] += jnp.sum(z, axis=0, keepdims=True)
    st_ref[1:2, :] += jnp.sum(z * z, axis=0, keepdims=True)

    @pl.when(i == _NBQ - 1)
    def _():
        zsum_ref[...] = st_ref[0:1, :]
        zsq_ref[...] = st_ref[1:2, :]


def _mid(query_feat, W_s0, scale_s, bias_s, qf_acc, wsum, W1, b1):
    return pl.pallas_call(
        _mid_body,
        grid=(_NBQ,),
        in_specs=[
            pl.BlockSpec((_BQ, D), lambda i: (i, 0)),
            pl.BlockSpec((D, D), lambda i: (0, 0)),
            pl.BlockSpec((1, D), lambda i: (0, 0)),
            pl.BlockSpec((1, D), lambda i: (0, 0)),
            pl.BlockSpec((_BQ, D), lambda i: (i, 0)),
            pl.BlockSpec((1, 1, _BQ), lambda i: (i, 0, 0)),
            pl.BlockSpec((D, D), lambda i: (0, 0)),
            pl.BlockSpec((1, D), lambda i: (0, 0)),
        ],
        out_specs=[
            pl.BlockSpec((_BQ, D), lambda i: (i, 0)),
            pl.BlockSpec((1, D), lambda i: (0, 0)),
            pl.BlockSpec((1, D), lambda i: (0, 0)),
        ],
        out_shape=[
            jax.ShapeDtypeStruct((N_Q, D), jnp.float32),
            jax.ShapeDtypeStruct((1, D), jnp.float32),
            jax.ShapeDtypeStruct((1, D), jnp.float32),
        ],
        scratch_shapes=[pltpu.VMEM((2, D), jnp.float32)],
    )(query_feat, W_s0.T, scale_s, bias_s, qf_acc,
      wsum.reshape(_NBQ, 1, _BQ), W1.T, b1.reshape(1, D))


def _fin_body(z_ref, sc_ref, bi_ref, o_ref):
    o_ref[...] = jnp.maximum(z_ref[...] * sc_ref[...] + bi_ref[...], 0.0)


def _final(z, scale_z, bias_z):
    return pl.pallas_call(
        _fin_body,
        grid=(_NBQ,),
        in_specs=[
            pl.BlockSpec((_BQ, D), lambda i: (i, 0)),
            pl.BlockSpec((1, D), lambda i: (0, 0)),
            pl.BlockSpec((1, D), lambda i: (0, 0)),
        ],
        out_specs=pl.BlockSpec((_BQ, D), lambda i: (i, 0)),
        out_shape=jax.ShapeDtypeStruct((N_Q, D), jnp.float32),
    )(z, scale_z, bias_z)


def kernel(ref_bxyz, query_bxyz, ref_feat, query_feat, e_ref, e_query,
           W_f0, gamma_f0, beta_f0, W_s0, gamma_s0, beta_s0,
           W1, b1, gamma1, beta1):
    feat2 = _ref_branch(ref_feat, W_f0, gamma_f0, beta_f0)
    qf_acc, wsum = _sc_edge_conv(ref_bxyz, query_bxyz, e_ref, e_query, feat2)
    ssum, ssq = _skip_stats(query_feat, W_s0)
    n = jnp.float32(N_Q)
    m_s = ssum / n
    v_s = ssq / n - m_s * m_s
    scale_s = gamma_s0.reshape(1, D) / jnp.sqrt(v_s + 1e-5)
    bias_s = beta_s0.reshape(1, D) - m_s * scale_s
    z, zsum, zsq = _mid(query_feat, W_s0, scale_s, bias_s, qf_acc, wsum, W1, b1)
    m_z = zsum / n
    v_z = zsq / n - m_z * m_z
    scale_z = gamma1.reshape(1, D) / jnp.sqrt(v_z + 1e-5)
    bias_z = beta1.reshape(1, D) - m_z * scale_z
    return _final(z, scale_z, bias_z)


# speedup vs baseline: 24.4365x; 1.0745x over previous
import functools

import jax
import jax.numpy as jnp
from jax import lax
from jax.experimental import pallas as pl
from jax.experimental.pallas import tpu as pltpu
from jax.experimental.pallas import tpu_sc as plsc

N_REF = 10000
N_Q = 100000
E = 300000
D = 128

NC, NS = 2, 16
NW = NC * NS
RW = 3136
NQ_PAD = NW * RW
WIN = 256
WSH = 8
KB = 128
EBLK = E // 16


def _rsqrt_nr(s):
    i = lax.bitcast_convert_type(s, jnp.int32)
    i = jnp.int32(0x5F3759DF) - (i >> 1)
    r = lax.bitcast_convert_type(i, jnp.float32)
    for _ in range(3):
        r = r * (1.5 - 0.5 * s * r * r)
    return r


def _sc_body(rx_h, ry_h, rz_h, qx_h, qy_h, qz_h, er_h, eq_h, feat_h, zero_h,
             qf_h, ws_h,
             rx_v, ry_v, rz_v, qx_v, qy_v, qz_v,
             acc_v, wsw_v, feat_a, feat_b, er_a, eq_a, er_b, eq_b,
             d_a, qc_a, d_b, qc_b, bs_v,
             sem_ga, sem_gb, sem_f, sem_w, sem_b):
    wid = lax.axis_index("c") * NS + lax.axis_index("s")
    row_lo = wid * RW
    row_hi = jnp.minimum(row_lo + RW, N_Q)

    def searchsorted(v):
        def body(_, c):
            lo, hi = c
            mid = (lo + hi) >> 1
            pltpu.async_copy(eq_h.at[pl.ds(pl.multiple_of(mid * 16, 16), 16)],
                             bs_v, sem_b).wait()
            below = bs_v[...][0] < v
            return jnp.where(below, mid, lo), jnp.where(below, hi, mid)

        lo, _ = lax.fori_loop(0, 15, body, (jnp.int32(0), jnp.int32(EBLK)))
        pltpu.async_copy(eq_h.at[pl.ds(pl.multiple_of(lo * 16, 16), 16)],
                         bs_v, sem_b).wait()
        cnt = jnp.sum((bs_v[...] < v).astype(jnp.int32))
        return lo * 16 + cnt

    e_lo = searchsorted(row_lo)
    e_hi = searchsorted(row_hi)
    a_lo = e_lo & jnp.int32(-8)

    row_lo8 = pl.multiple_of(row_lo, 64)
    pltpu.async_copy(rx_h, rx_v, sem_b)
    pltpu.async_copy(ry_h, ry_v, sem_b)
    pltpu.async_copy(rz_h, rz_v, sem_b)
    pltpu.async_copy(qx_h.at[pl.ds(row_lo8, RW)], qx_v, sem_b)
    pltpu.async_copy(qy_h.at[pl.ds(row_lo8, RW)], qy_v, sem_b)
    pltpu.async_copy(qz_h.at[pl.ds(row_lo8, RW)], qz_v, sem_b)
    pltpu.async_copy(zero_h, acc_v, sem_b)
    pltpu.async_copy(zero_h.at[pl.ds(0, WIN)], wsw_v, sem_b)
    pltpu.make_async_copy(rx_h, rx_v, sem_b).wait()
    pltpu.make_async_copy(ry_h, ry_v, sem_b).wait()
    pltpu.make_async_copy(rz_h, rz_v, sem_b).wait()
    pltpu.make_async_copy(qx_h.at[pl.ds(row_lo8, RW)], qx_v, sem_b).wait()
    pltpu.make_async_copy(qy_h.at[pl.ds(row_lo8, RW)], qy_v, sem_b).wait()
    pltpu.make_async_copy(qz_h.at[pl.ds(row_lo8, RW)], qz_v, sem_b).wait()
    pltpu.make_async_copy(zero_h, acc_v, sem_b).wait()
    pltpu.make_async_copy(zero_h.at[pl.ds(0, WIN)], wsw_v, sem_b).wait()

    def flush(wb, nrows):
        n32 = nrows >> 5

        def issue(t, _):
            o1 = pl.multiple_of((wb + t * 32) * D, 4096)
            o2 = pl.multiple_of(wb + t * 32, 32)
            pltpu.async_copy(acc_v.at[pl.ds(t * 4096, 4096)],
                             qf_h.at[pl.ds(o1, 4096)], sem_f)
            pltpu.async_copy(wsw_v.at[pl.ds(t * 32, 32)],
                             ws_h.at[pl.ds(o2, 32)], sem_w)
            return 0

        def drain(t, _):
            o1 = pl.multiple_of((wb + t * 32) * D, 4096)
            o2 = pl.multiple_of(wb + t * 32, 32)
            pltpu.make_async_copy(acc_v.at[pl.ds(t * 4096, 4096)],
                                  qf_h.at[pl.ds(o1, 4096)],
                                  sem_f).wait()
            pltpu.make_async_copy(wsw_v.at[pl.ds(t * 32, 32)],
                                  ws_h.at[pl.ds(o2, 32)],
                                  sem_w).wait()
            return 0

        lax.fori_loop(0, n32, issue, 0)
        lax.fori_loop(0, n32, drain, 0)
        pltpu.async_copy(zero_h, acc_v, sem_b)
        pltpu.async_copy(zero_h.at[pl.ds(0, WIN)], wsw_v, sem_b)
        pltpu.make_async_copy(zero_h, acc_v, sem_b).wait()
        pltpu.make_async_copy(zero_h.at[pl.ds(0, WIN)], wsw_v, sem_b).wait()

    def advance(wb):
        flush(wb, jnp.minimum(WIN, row_hi - wb))
        return wb + WIN

    lane = lax.iota(jnp.int32, 16)
    lane0 = lane == 0

    def groups(feat_v, d_v, qc_v, wb):
        def group_body(g, wb):
            eqb = qc_v[pl.ds(16 * g, 16)]
            db = d_v[pl.ds(16 * g, 16)]
            q0 = eqb[0]
            q15 = eqb[15]
            wb = lax.fori_loop(0, (q0 - wb) >> WSH,
                               lambda _, w: advance(w), wb)

            def fast(wb):
                for k in range(16):
                    ro = eqb[k] - wb
                    dv = jnp.full((16,), db[k])
                    for j in range(D // 16):
                        f = feat_v[16 * g + k, pl.ds(16 * j, 16)]
                        plsc.addupdate(acc_v.at[pl.ds(ro * D + 16 * j, 16)],
                                       f * dv)
                    plsc.addupdate_scatter(wsw_v,
                                           [jnp.full((16,), ro, jnp.int32)],
                                           db, mask=lane == k)
                return wb

            def slow(wb):
                def eb(k, w):
                    iv = jnp.full((16,), 16 * g + k, jnp.int32)
                    q = plsc.load_gather(qc_v, [iv])[0]
                    dbv = plsc.load_gather(d_v, [iv])
                    w = lax.fori_loop(0, (q - w) >> WSH,
                                      lambda _, x: advance(x), w)
                    ro = q - w
                    for j in range(D // 16):
                        f = feat_v[16 * g + k, pl.ds(16 * j, 16)]
                        plsc.addupdate(acc_v.at[pl.ds(ro * D + 16 * j, 16)],
                                       f * dbv)
                    plsc.addupdate_scatter(wsw_v,
                                           [jnp.full((16,), ro, jnp.int32)],
                                           dbv, mask=lane0)
                    return w

                return lax.fori_loop(0, 16, eb, wb)

            return lax.cond(q15 - wb < WIN, fast, slow, wb)

        return lax.fori_loop(0, KB // 16, group_body, wb)

    def load_idx(b, er_v, eq_v):
        base = pl.multiple_of(a_lo + b * KB, 8)
        pltpu.async_copy(er_h.at[pl.ds(base, KB)], er_v, sem_b)
        pltpu.async_copy(eq_h.at[pl.ds(base, KB)], eq_v, sem_b)
        pltpu.make_async_copy(er_h.at[pl.ds(base, KB)], er_v, sem_b).wait()
        pltpu.make_async_copy(eq_h.at[pl.ds(base, KB)], eq_v, sem_b).wait()

    def weights(er_v, eq_v, d_v, qc_v):
        for g in range(KB // 16):
            qi = eq_v[pl.ds(16 * g, 16)]
            ri = er_v[pl.ds(16 * g, 16)]
            valid = (qi >= row_lo) & (qi < row_hi)
            qidx = jnp.where(valid, qi - row_lo, 0)
            qx = plsc.load_gather(qx_v, [qidx], mask=valid)
            qy = plsc.load_gather(qy_v, [qidx], mask=valid)
            qz = plsc.load_gather(qz_v, [qidx], mask=valid)
            rxx = plsc.load_gather(rx_v, [ri])
            ryy = plsc.load_gather(ry_v, [ri])
            rzz = plsc.load_gather(rz_v, [ri])
            dx = rxx - qx
            dy = ryy - qy
            dz = rzz - qz
            s = jnp.maximum(dx * dx + dy * dy + dz * dz, 1e-30)
            r = _rsqrt_nr(s)
            dd = r / (1.0 + 1e-8 * r)
            d_v[pl.ds(16 * g, 16)] = jnp.where(valid, dd, 0.0)
            qc_v[pl.ds(16 * g, 16)] = jnp.minimum(
                jnp.maximum(qi, row_lo), row_hi - 1)

    nblk = (e_hi - a_lo + (KB - 1)) >> 7
    npair = jnp.maximum((nblk + 1) >> 1, 1)
    load_idx(0, er_a, eq_a)
    pltpu.async_copy(feat_h.at[er_a], feat_a, sem_ga)

    def pair_body(i, wb):
        load_idx(2 * i + 1, er_b, eq_b)
        pltpu.async_copy(feat_h.at[er_b], feat_b, sem_gb)
        weights(er_a, eq_a, d_a, qc_a)
        pltpu.make_async_copy(feat_h.at[er_a], feat_a, sem_ga).wait()
        wb = groups(feat_a, d_a, qc_a, wb)
        load_idx(2 * i + 2, er_a, eq_a)
        pltpu.async_copy(feat_h.at[er_a], feat_a, sem_ga)
        weights(er_b, eq_b, d_b, qc_b)
        pltpu.make_async_copy(feat_h.at[er_b], feat_b, sem_gb).wait()
        wb = groups(feat_b, d_b, qc_b, wb)
        return wb

    wb = lax.fori_loop(0, npair, pair_body, row_lo)
    pltpu.make_async_copy(feat_h.at[er_a], feat_a, sem_ga).wait()
    lax.fori_loop(0, (row_hi - wb + (WIN - 1)) >> WSH,
                  lambda _, w: advance(w), wb)


def _sc_edge_conv(ref_bxyz, query_bxyz, e_ref, e_query, feat):
    mesh = plsc.VectorSubcoreMesh(core_axis_name="c", subcore_axis_name="s")
    rx = ref_bxyz[:, 1] + 0.0
    ry = ref_bxyz[:, 2] + 0.0
    rz = ref_bxyz[:, 3] + 0.0
    qpad = jnp.pad(query_bxyz[:, 1:4], ((0, NQ_PAD - N_Q), (0, 0)))
    qx = qpad[:, 0] + 0.0
    qy = qpad[:, 1] + 0.0
    qz = qpad[:, 2] + 0.0
    er_p = jnp.pad(e_ref, (0, 3 * KB))
    eq_p = jnp.pad(e_query, (0, 3 * KB), constant_values=N_Q)
    zeros = jnp.zeros((WIN * D,), jnp.float32)
    run = pl.kernel(
        _sc_body,
        out_type=(jax.ShapeDtypeStruct((N_Q * D,), jnp.float32),
                  jax.ShapeDtypeStruct((N_Q,), jnp.float32)),
        mesh=mesh,
        compiler_params=pltpu.CompilerParams(needs_layout_passes=False),
        scratch_types=[
            pltpu.VMEM((N_REF,), jnp.float32),
            pltpu.VMEM((N_REF,), jnp.float32),
            pltpu.VMEM((N_REF,), jnp.float32),
            pltpu.VMEM((RW,), jnp.float32),
            pltpu.VMEM((RW,), jnp.float32),
            pltpu.VMEM((RW,), jnp.float32),
            pltpu.VMEM((WIN * D,), jnp.float32),
            pltpu.VMEM((WIN,), jnp.float32),
            pltpu.VMEM((KB, D), jnp.float32),
            pltpu.VMEM((KB, D), jnp.float32),
            pltpu.VMEM((KB,), jnp.int32),
            pltpu.VMEM((KB,), jnp.int32),
            pltpu.VMEM((KB,), jnp.int32),
            pltpu.VMEM((KB,), jnp.int32),
            pltpu.VMEM((KB,), jnp.float32),
            pltpu.VMEM((KB,), jnp.int32),
            pltpu.VMEM((KB,), jnp.float32),
            pltpu.VMEM((KB,), jnp.int32),
            pltpu.VMEM((16,), jnp.int32),
            pltpu.SemaphoreType.DMA,
            pltpu.SemaphoreType.DMA,
            pltpu.SemaphoreType.DMA,
            pltpu.SemaphoreType.DMA,
            pltpu.SemaphoreType.DMA,
        ],
    )
    qf_flat, wsum = run(rx, ry, rz, qx, qy, qz, er_p, eq_p, feat, zeros)
    return qf_flat.reshape(N_Q, D), wsum


def _k1_body(x_ref, wt_ref, g_ref, b_ref, o_ref):
    y = jnp.dot(x_ref[...], wt_ref[...], preferred_element_type=jnp.float32)
    m = jnp.mean(y, axis=0, keepdims=True)
    v = jnp.mean((y - m) ** 2, axis=0, keepdims=True)
    o_ref[...] = g_ref[...] * (y - m) / jnp.sqrt(v + 1e-5) + b_ref[...]


def _ref_branch(ref_feat, W_f0, gamma, beta):
    return pl.pallas_call(
        _k1_body,
        out_shape=jax.ShapeDtypeStruct((N_REF, D), jnp.float32),
    )(ref_feat, W_f0.T, gamma.reshape(1, D), beta.reshape(1, D))


_BQ = 1000
_NBQ = N_Q // _BQ


def _stats_body(x_ref, wt_ref, ssum_ref, ssq_ref, acc_ref):
    i = pl.program_id(0)

    @pl.when(i == 0)
    def _():
        acc_ref[...] = jnp.zeros_like(acc_ref)

    y = jnp.dot(x_ref[...], wt_ref[...], preferred_element_type=jnp.float32)
    acc_ref[0:1, :] += jnp.sum(y, axis=0, keepdims=True)
    acc_ref[1:2, :] += jnp.sum(y * y, axis=0, keepdims=True)

    @pl.when(i == _NBQ - 1)
    def _():
        ssum_ref[...] = acc_ref[0:1, :]
        ssq_ref[...] = acc_ref[1:2, :]


def _skip_stats(query_feat, W_s0):
    return pl.pallas_call(
        _stats_body,
        grid=(_NBQ,),
        in_specs=[
            pl.BlockSpec((_BQ, D), lambda i: (i, 0)),
            pl.BlockSpec((D, D), lambda i: (0, 0)),
        ],
        out_specs=[
            pl.BlockSpec((1, D), lambda i: (0, 0)),
            pl.BlockSpec((1, D), lambda i: (0, 0)),
        ],
        out_shape=[
            jax.ShapeDtypeStruct((1, D), jnp.float32),
            jax.ShapeDtypeStruct((1, D), jnp.float32),
        ],
        scratch_shapes=[pltpu.VMEM((2, D), jnp.float32)],
    )(query_feat, W_s0.T)


def _mid_body(x_ref, wt_ref, sc_ref, bi_ref, acc_ref, w_ref, w1t_ref, b1_ref,
              z_ref, zsum_ref, zsq_ref, st_ref):
    i = pl.program_id(0)

    @pl.when(i == 0)
    def _():
        st_ref[...] = jnp.zeros_like(st_ref)

    ys = jnp.dot(x_ref[...], wt_ref[...], preferred_element_type=jnp.float32)
    skip = ys * sc_ref[...] + bi_ref[...]
    w = w_ref[...].reshape(_BQ, 1)
    inv = jnp.where(w > 0, 1.0 / w, 0.0)
    h = jnp.maximum(acc_ref[...] * inv + skip, 0.0)
    z = jnp.dot(h, w1t_ref[...], preferred_element_type=jnp.float32) + b1_ref[...]
    z_ref[...] = z
    st_ref[0:1, :] += jnp.sum(z, axis=0, keepdims=True)
    st_ref[1:2, :] += jnp.sum(z * z, axis=0, keepdims=True)

    @pl.when(i == _NBQ - 1)
    def _():
        zsum_ref[...] = st_ref[0:1, :]
        zsq_ref[...] = st_ref[1:2, :]


def _mid(query_feat, W_s0, scale_s, bias_s, qf_acc, wsum, W1, b1):
    return pl.pallas_call(
        _mid_body,
        grid=(_NBQ,),
        in_specs=[
            pl.BlockSpec((_BQ, D), lambda i: (i, 0)),
            pl.BlockSpec((D, D), lambda i: (0, 0)),
            pl.BlockSpec((1, D), lambda i: (0, 0)),
            pl.BlockSpec((1, D), lambda i: (0, 0)),
            pl.BlockSpec((_BQ, D), lambda i: (i, 0)),
            pl.BlockSpec((1, 1, _BQ), lambda i: (i, 0, 0)),
            pl.BlockSpec((D, D), lambda i: (0, 0)),
            pl.BlockSpec((1, D), lambda i: (0, 0)),
        ],
        out_specs=[
            pl.BlockSpec((_BQ, D), lambda i: (i, 0)),
            pl.BlockSpec((1, D), lambda i: (0, 0)),
            pl.BlockSpec((1, D), lambda i: (0, 0)),
        ],
        out_shape=[
            jax.ShapeDtypeStruct((N_Q, D), jnp.float32),
            jax.ShapeDtypeStruct((1, D), jnp.float32),
            jax.ShapeDtypeStruct((1, D), jnp.float32),
        ],
        scratch_shapes=[pltpu.VMEM((2, D), jnp.float32)],
    )(query_feat, W_s0.T, scale_s, bias_s, qf_acc,
      wsum.reshape(_NBQ, 1, _BQ), W1.T, b1.reshape(1, D))


def _fin_body(z_ref, sc_ref, bi_ref, o_ref):
    o_ref[...] = jnp.maximum(z_ref[...] * sc_ref[...] + bi_ref[...], 0.0)


def _final(z, scale_z, bias_z):
    return pl.pallas_call(
        _fin_body,
        grid=(_NBQ,),
        in_specs=[
            pl.BlockSpec((_BQ, D), lambda i: (i, 0)),
            pl.BlockSpec((1, D), lambda i: (0, 0)),
            pl.BlockSpec((1, D), lambda i: (0, 0)),
        ],
        out_specs=pl.BlockSpec((_BQ, D), lambda i: (i, 0)),
        out_shape=jax.ShapeDtypeStruct((N_Q, D), jnp.float32),
    )(z, scale_z, bias_z)


def kernel(ref_bxyz, query_bxyz, ref_feat, query_feat, e_ref, e_query,
           W_f0, gamma_f0, beta_f0, W_s0, gamma_s0, beta_s0,
           W1, b1, gamma1, beta1):
    feat2 = _ref_branch(ref_feat, W_f0, gamma_f0, beta_f0)
    qf_acc, wsum = _sc_edge_conv(ref_bxyz, query_bxyz, e_ref, e_query, feat2)
    ssum, ssq = _skip_stats(query_feat, W_s0)
    n = jnp.float32(N_Q)
    m_s = ssum / n
    v_s = ssq / n - m_s * m_s
    scale_s = gamma_s0.reshape(1, D) / jnp.sqrt(v_s + 1e-5)
    bias_s = beta_s0.reshape(1, D) - m_s * scale_s
    z, zsum, zsq = _mid(query_feat, W_s0, scale_s, bias_s, qf_acc, wsum, W1, b1)
    m_z = zsum / n
    v_z = zsq / n - m_z * m_z
    scale_z = gamma1.reshape(1, D) / jnp.sqrt(v_z + 1e-5)
    bias_z = beta1.reshape(1, D) - m_z * scale_z
    return _final(z, scale_z, bias_z)


# speedup vs baseline: 24.4863x; 1.0020x over previous
import functools

import jax
import jax.numpy as jnp
from jax import lax
from jax.experimental import pallas as pl
from jax.experimental.pallas import tpu as pltpu
from jax.experimental.pallas import tpu_sc as plsc

N_REF = 10000
N_Q = 100000
E = 300000
D = 128

NC, NS = 2, 16
NW = NC * NS
RW = 3136
NQ_PAD = NW * RW
WIN = 256
WSH = 8
KB = 128
EBLK = E // 16


def _rsqrt_nr(s):
    i = lax.bitcast_convert_type(s, jnp.int32)
    i = jnp.int32(0x5F3759DF) - (i >> 1)
    r = lax.bitcast_convert_type(i, jnp.float32)
    for _ in range(3):
        r = r * (1.5 - 0.5 * s * r * r)
    return r


def _sc_body(rx_h, ry_h, rz_h, qx_h, qy_h, qz_h, er_h, eq_h, eb_h, feat_h,
             zero_h, qf_h, ws_h,
             rx_v, ry_v, rz_v, qx_v, qy_v, qz_v,
             acc_v, wsw_v, feat_a, feat_b, er_a, eq_a, er_b, eq_b,
             d_a, qc_a, d_b, qc_b, eb_v,
             sem_ga, sem_gb, sem_f, sem_w, sem_b):
    wid = lax.axis_index("c") * NS + lax.axis_index("s")
    row_lo = wid * RW
    row_hi = jnp.minimum(row_lo + RW, N_Q)

    pltpu.async_copy(eb_h, eb_v, sem_ga)
    row_lo8 = pl.multiple_of(row_lo, 64)
    pltpu.async_copy(rx_h, rx_v, sem_b)
    pltpu.async_copy(ry_h, ry_v, sem_b)
    pltpu.async_copy(rz_h, rz_v, sem_b)
    pltpu.async_copy(qx_h.at[pl.ds(row_lo8, RW)], qx_v, sem_b)
    pltpu.async_copy(qy_h.at[pl.ds(row_lo8, RW)], qy_v, sem_b)
    pltpu.async_copy(qz_h.at[pl.ds(row_lo8, RW)], qz_v, sem_b)
    pltpu.async_copy(zero_h, acc_v, sem_b)
    pltpu.async_copy(zero_h.at[pl.ds(0, WIN)], wsw_v, sem_b)

    pltpu.make_async_copy(eb_h, eb_v, sem_ga).wait()
    widv = jnp.full((16,), wid, jnp.int32)
    e_lo = plsc.load_gather(eb_v, [widv])[0]
    e_hi = plsc.load_gather(eb_v, [widv + 1])[0]
    a_lo = e_lo & jnp.int32(-8)

    pltpu.make_async_copy(rx_h, rx_v, sem_b).wait()
    pltpu.make_async_copy(ry_h, ry_v, sem_b).wait()
    pltpu.make_async_copy(rz_h, rz_v, sem_b).wait()
    pltpu.make_async_copy(qx_h.at[pl.ds(row_lo8, RW)], qx_v, sem_b).wait()
    pltpu.make_async_copy(qy_h.at[pl.ds(row_lo8, RW)], qy_v, sem_b).wait()
    pltpu.make_async_copy(qz_h.at[pl.ds(row_lo8, RW)], qz_v, sem_b).wait()
    pltpu.make_async_copy(zero_h, acc_v, sem_b).wait()
    pltpu.make_async_copy(zero_h.at[pl.ds(0, WIN)], wsw_v, sem_b).wait()

    def flush(wb, nrows):
        n32 = nrows >> 5

        def issue(t, _):
            o1 = pl.multiple_of((wb + t * 32) * D, 4096)
            o2 = pl.multiple_of(wb + t * 32, 32)
            pltpu.async_copy(acc_v.at[pl.ds(t * 4096, 4096)],
                             qf_h.at[pl.ds(o1, 4096)], sem_f)
            pltpu.async_copy(wsw_v.at[pl.ds(t * 32, 32)],
                             ws_h.at[pl.ds(o2, 32)], sem_w)
            return 0

        def drain(t, _):
            o1 = pl.multiple_of((wb + t * 32) * D, 4096)
            o2 = pl.multiple_of(wb + t * 32, 32)
            pltpu.make_async_copy(acc_v.at[pl.ds(t * 4096, 4096)],
                                  qf_h.at[pl.ds(o1, 4096)],
                                  sem_f).wait()
            pltpu.make_async_copy(wsw_v.at[pl.ds(t * 32, 32)],
                                  ws_h.at[pl.ds(o2, 32)],
                                  sem_w).wait()
            return 0

        lax.fori_loop(0, n32, issue, 0)
        lax.fori_loop(0, n32, drain, 0)
        pltpu.async_copy(zero_h, acc_v, sem_b)
        pltpu.async_copy(zero_h.at[pl.ds(0, WIN)], wsw_v, sem_b)
        pltpu.make_async_copy(zero_h, acc_v, sem_b).wait()
        pltpu.make_async_copy(zero_h.at[pl.ds(0, WIN)], wsw_v, sem_b).wait()

    def advance(wb):
        flush(wb, jnp.minimum(WIN, row_hi - wb))
        return wb + WIN

    lane = lax.iota(jnp.int32, 16)
    lane0 = lane == 0

    def groups(feat_v, d_v, qc_v, wb):
        def group_body(g, wb):
            eqb = qc_v[pl.ds(16 * g, 16)]
            db = d_v[pl.ds(16 * g, 16)]
            q0 = eqb[0]
            q15 = eqb[15]
            wb = lax.fori_loop(0, (q0 - wb) >> WSH,
                               lambda _, w: advance(w), wb)

            def fast(wb):
                for k in range(16):
                    ro = eqb[k] - wb
                    dv = jnp.full((16,), db[k])
                    for j in range(D // 16):
                        f = feat_v[16 * g + k, pl.ds(16 * j, 16)]
                        plsc.addupdate(acc_v.at[pl.ds(ro * D + 16 * j, 16)],
                                       f * dv)
                    plsc.addupdate_scatter(wsw_v,
                                           [jnp.full((16,), ro, jnp.int32)],
                                           db, mask=lane == k)
                return wb

            def slow(wb):
                def eb(k, w):
                    iv = jnp.full((16,), 16 * g + k, jnp.int32)
                    q = plsc.load_gather(qc_v, [iv])[0]
                    dbv = plsc.load_gather(d_v, [iv])
                    w = lax.fori_loop(0, (q - w) >> WSH,
                                      lambda _, x: advance(x), w)
                    ro = q - w
                    for j in range(D // 16):
                        f = feat_v[16 * g + k, pl.ds(16 * j, 16)]
                        plsc.addupdate(acc_v.at[pl.ds(ro * D + 16 * j, 16)],
                                       f * dbv)
                    plsc.addupdate_scatter(wsw_v,
                                           [jnp.full((16,), ro, jnp.int32)],
                                           dbv, mask=lane0)
                    return w

                return lax.fori_loop(0, 16, eb, wb)

            return lax.cond(q15 - wb < WIN, fast, slow, wb)

        return lax.fori_loop(0, KB // 16, group_body, wb)

    def load_idx(b, er_v, eq_v):
        base = pl.multiple_of(a_lo + b * KB, 8)
        pltpu.async_copy(er_h.at[pl.ds(base, KB)], er_v, sem_b)
        pltpu.async_copy(eq_h.at[pl.ds(base, KB)], eq_v, sem_b)
        pltpu.make_async_copy(er_h.at[pl.ds(base, KB)], er_v, sem_b).wait()
        pltpu.make_async_copy(eq_h.at[pl.ds(base, KB)], eq_v, sem_b).wait()

    def weights(er_v, eq_v, d_v, qc_v):
        for g in range(KB // 16):
            qi = eq_v[pl.ds(16 * g, 16)]
            ri = er_v[pl.ds(16 * g, 16)]
            valid = (qi >= row_lo) & (qi < row_hi)
            qidx = jnp.where(valid, qi - row_lo, 0)
            qx = plsc.load_gather(qx_v, [qidx], mask=valid)
            qy = plsc.load_gather(qy_v, [qidx], mask=valid)
            qz = plsc.load_gather(qz_v, [qidx], mask=valid)
            rxx = plsc.load_gather(rx_v, [ri])
            ryy = plsc.load_gather(ry_v, [ri])
            rzz = plsc.load_gather(rz_v, [ri])
            dx = rxx - qx
            dy = ryy - qy
            dz = rzz - qz
            s = jnp.maximum(dx * dx + dy * dy + dz * dz, 1e-30)
            r = _rsqrt_nr(s)
            dd = r / (1.0 + 1e-8 * r)
            d_v[pl.ds(16 * g, 16)] = jnp.where(valid, dd, 0.0)
            qc_v[pl.ds(16 * g, 16)] = jnp.minimum(
                jnp.maximum(qi, row_lo), row_hi - 1)

    nblk = (e_hi - a_lo + (KB - 1)) >> 7
    npair = jnp.maximum((nblk + 1) >> 1, 1)
    load_idx(0, er_a, eq_a)
    pltpu.async_copy(feat_h.at[er_a], feat_a, sem_ga)

    def pair_body(i, wb):
        load_idx(2 * i + 1, er_b, eq_b)
        pltpu.async_copy(feat_h.at[er_b], feat_b, sem_gb)
        weights(er_a, eq_a, d_a, qc_a)
        pltpu.make_async_copy(feat_h.at[er_a], feat_a, sem_ga).wait()
        wb = groups(feat_a, d_a, qc_a, wb)
        load_idx(2 * i + 2, er_a, eq_a)
        pltpu.async_copy(feat_h.at[er_a], feat_a, sem_ga)
        weights(er_b, eq_b, d_b, qc_b)
        pltpu.make_async_copy(feat_h.at[er_b], feat_b, sem_gb).wait()
        wb = groups(feat_b, d_b, qc_b, wb)
        return wb

    wb = lax.fori_loop(0, npair, pair_body, row_lo)
    pltpu.make_async_copy(feat_h.at[er_a], feat_a, sem_ga).wait()
    lax.fori_loop(0, (row_hi - wb + (WIN - 1)) >> WSH,
                  lambda _, w: advance(w), wb)


def _sc_edge_conv(ref_bxyz, query_bxyz, e_ref, e_query, feat):
    mesh = plsc.VectorSubcoreMesh(core_axis_name="c", subcore_axis_name="s")
    rx = ref_bxyz[:, 1] + 0.0
    ry = ref_bxyz[:, 2] + 0.0
    rz = ref_bxyz[:, 3] + 0.0
    qpad = jnp.pad(query_bxyz[:, 1:4], ((0, NQ_PAD - N_Q), (0, 0)))
    qx = qpad[:, 0] + 0.0
    qy = qpad[:, 1] + 0.0
    qz = qpad[:, 2] + 0.0
    er_p = jnp.pad(e_ref, (0, 3 * KB))
    eq_p = jnp.pad(e_query, (0, 3 * KB), constant_values=N_Q)
    bnd = jnp.minimum(jnp.arange(NW + 1, dtype=jnp.int32) * RW, N_Q)
    ebnd = jnp.searchsorted(e_query, bnd).astype(jnp.int32)
    ebnd = jnp.pad(ebnd, (0, 48 - (NW + 1)))
    zeros = jnp.zeros((WIN * D,), jnp.float32)
    run = pl.kernel(
        _sc_body,
        out_type=(jax.ShapeDtypeStruct((N_Q * D,), jnp.float32),
                  jax.ShapeDtypeStruct((N_Q,), jnp.float32)),
        mesh=mesh,
        compiler_params=pltpu.CompilerParams(needs_layout_passes=False),
        scratch_types=[
            pltpu.VMEM((N_REF,), jnp.float32),
            pltpu.VMEM((N_REF,), jnp.float32),
            pltpu.VMEM((N_REF,), jnp.float32),
            pltpu.VMEM((RW,), jnp.float32),
            pltpu.VMEM((RW,), jnp.float32),
            pltpu.VMEM((RW,), jnp.float32),
            pltpu.VMEM((WIN * D,), jnp.float32),
            pltpu.VMEM((WIN,), jnp.float32),
            pltpu.VMEM((KB, D), jnp.float32),
            pltpu.VMEM((KB, D), jnp.float32),
            pltpu.VMEM((KB,), jnp.int32),
            pltpu.VMEM((KB,), jnp.int32),
            pltpu.VMEM((KB,), jnp.int32),
            pltpu.VMEM((KB,), jnp.int32),
            pltpu.VMEM((KB,), jnp.float32),
            pltpu.VMEM((KB,), jnp.int32),
            pltpu.VMEM((KB,), jnp.float32),
            pltpu.VMEM((KB,), jnp.int32),
            pltpu.VMEM((48,), jnp.int32),
            pltpu.SemaphoreType.DMA,
            pltpu.SemaphoreType.DMA,
            pltpu.SemaphoreType.DMA,
            pltpu.SemaphoreType.DMA,
            pltpu.SemaphoreType.DMA,
        ],
    )
    qf_flat, wsum = run(rx, ry, rz, qx, qy, qz, er_p, eq_p, ebnd, feat, zeros)
    return qf_flat.reshape(N_Q, D), wsum


def _k1_body(x_ref, wt_ref, g_ref, b_ref, o_ref):
    y = jnp.dot(x_ref[...], wt_ref[...], preferred_element_type=jnp.float32)
    m = jnp.mean(y, axis=0, keepdims=True)
    v = jnp.mean((y - m) ** 2, axis=0, keepdims=True)
    o_ref[...] = g_ref[...] * (y - m) / jnp.sqrt(v + 1e-5) + b_ref[...]


def _ref_branch(ref_feat, W_f0, gamma, beta):
    return pl.pallas_call(
        _k1_body,
        out_shape=jax.ShapeDtypeStruct((N_REF, D), jnp.float32),
    )(ref_feat, W_f0.T, gamma.reshape(1, D), beta.reshape(1, D))


_BQ = 1000
_NBQ = N_Q // _BQ


def _stats_body(x_ref, wt_ref, ssum_ref, ssq_ref, acc_ref):
    i = pl.program_id(0)

    @pl.when(i == 0)
    def _():
        acc_ref[...] = jnp.zeros_like(acc_ref)

    y = jnp.dot(x_ref[...], wt_ref[...], preferred_element_type=jnp.float32)
    acc_ref[0:1, :] += jnp.sum(y, axis=0, keepdims=True)
    acc_ref[1:2, :] += jnp.sum(y * y, axis=0, keepdims=True)

    @pl.when(i == _NBQ - 1)
    def _():
        ssum_ref[...] = acc_ref[0:1, :]
        ssq_ref[...] = acc_ref[1:2, :]


def _skip_stats(query_feat, W_s0):
    return pl.pallas_call(
        _stats_body,
        grid=(_NBQ,),
        in_specs=[
            pl.BlockSpec((_BQ, D), lambda i: (i, 0)),
            pl.BlockSpec((D, D), lambda i: (0, 0)),
        ],
        out_specs=[
            pl.BlockSpec((1, D), lambda i: (0, 0)),
            pl.BlockSpec((1, D), lambda i: (0, 0)),
        ],
        out_shape=[
            jax.ShapeDtypeStruct((1, D), jnp.float32),
            jax.ShapeDtypeStruct((1, D), jnp.float32),
        ],
        scratch_shapes=[pltpu.VMEM((2, D), jnp.float32)],
    )(query_feat, W_s0.T)


def _mid_body(x_ref, wt_ref, sc_ref, bi_ref, acc_ref, w_ref, w1t_ref, b1_ref,
              z_ref, zsum_ref, zsq_ref, st_ref):
    i = pl.program_id(0)

    @pl.when(i == 0)
    def _():
        st_ref[...] = jnp.zeros_like(st_ref)

    ys = jnp.dot(x_ref[...], wt_ref[...], preferred_element_type=jnp.float32)
    skip = ys * sc_ref[...] + bi_ref[...]
    w = w_ref[...].reshape(_BQ, 1)
    inv = jnp.where(w > 0, 1.0 / w, 0.0)
    h = jnp.maximum(acc_ref[...] * inv + skip, 0.0)
    z = jnp.dot(h, w1t_ref[...], preferred_element_type=jnp.float32) + b1_ref[...]
    z_ref[...] = z
    st_ref[0:1, :] += jnp.sum(z, axis=0, keepdims=True)
    st_ref[1:2, :] += jnp.sum(z * z, axis=0, keepdims=True)

    @pl.when(i == _NBQ - 1)
    def _():
        zsum_ref[...] = st_ref[0:1, :]
        zsq_ref[...] = st_ref[1:2, :]


def _mid(query_feat, W_s0, scale_s, bias_s, qf_acc, wsum, W1, b1):
    return pl.pallas_call(
        _mid_body,
        grid=(_NBQ,),
        in_specs=[
            pl.BlockSpec((_BQ, D), lambda i: (i, 0)),
            pl.BlockSpec((D, D), lambda i: (0, 0)),
            pl.BlockSpec((1, D), lambda i: (0, 0)),
            pl.BlockSpec((1, D), lambda i: (0, 0)),
            pl.BlockSpec((_BQ, D), lambda i: (i, 0)),
            pl.BlockSpec((1, 1, _BQ), lambda i: (i, 0, 0)),
            pl.BlockSpec((D, D), lambda i: (0, 0)),
            pl.BlockSpec((1, D), lambda i: (0, 0)),
        ],
        out_specs=[
            pl.BlockSpec((_BQ, D), lambda i: (i, 0)),
            pl.BlockSpec((1, D), lambda i: (0, 0)),
            pl.BlockSpec((1, D), lambda i: (0, 0)),
        ],
        out_shape=[
            jax.ShapeDtypeStruct((N_Q, D), jnp.float32),
            jax.ShapeDtypeStruct((1, D), jnp.float32),
            jax.ShapeDtypeStruct((1, D), jnp.float32),
        ],
        scratch_shapes=[pltpu.VMEM((2, D), jnp.float32)],
    )(query_feat, W_s0.T, scale_s, bias_s, qf_acc,
      wsum.reshape(_NBQ, 1, _BQ), W1.T, b1.reshape(1, D))


def _fin_body(z_ref, sc_ref, bi_ref, o_ref):
    o_ref[...] = jnp.maximum(z_ref[...] * sc_ref[...] + bi_ref[...], 0.0)


def _final(z, scale_z, bias_z):
    return pl.pallas_call(
        _fin_body,
        grid=(_NBQ,),
        in_specs=[
            pl.BlockSpec((_BQ, D), lambda i: (i, 0)),
            pl.BlockSpec((1, D), lambda i: (0, 0)),
            pl.BlockSpec((1, D), lambda i: (0, 0)),
        ],
        out_specs=pl.BlockSpec((_BQ, D), lambda i: (i, 0)),
        out_shape=jax.ShapeDtypeStruct((N_Q, D), jnp.float32),
    )(z, scale_z, bias_z)


def kernel(ref_bxyz, query_bxyz, ref_feat, query_feat, e_ref, e_query,
           W_f0, gamma_f0, beta_f0, W_s0, gamma_s0, beta_s0,
           W1, b1, gamma1, beta1):
    feat2 = _ref_branch(ref_feat, W_f0, gamma_f0, beta_f0)
    qf_acc, wsum = _sc_edge_conv(ref_bxyz, query_bxyz, e_ref, e_query, feat2)
    ssum, ssq = _skip_stats(query_feat, W_s0)
    n = jnp.float32(N_Q)
    m_s = ssum / n
    v_s = ssq / n - m_s * m_s
    scale_s = gamma_s0.reshape(1, D) / jnp.sqrt(v_s + 1e-5)
    bias_s = beta_s0.reshape(1, D) - m_s * scale_s
    z, zsum, zsq = _mid(query_feat, W_s0, scale_s, bias_s, qf_acc, wsum, W1, b1)
    m_z = zsum / n
    v_z = zsq / n - m_z * m_z
    scale_z = gamma1.reshape(1, D) / jnp.sqrt(v_z + 1e-5)
    bias_z = beta1.reshape(1, D) - m_z * scale_z
    return _final(z, scale_z, bias_z)


# speedup vs baseline: 25.7129x; 1.0501x over previous
import functools

import jax
import jax.numpy as jnp
from jax import lax
from jax.experimental import pallas as pl
from jax.experimental.pallas import tpu as pltpu
from jax.experimental.pallas import tpu_sc as plsc

N_REF = 10000
N_Q = 100000
E = 300000
D = 128

NC, NS = 2, 16
NW = NC * NS
RW = 3136
NQ_PAD = NW * RW
WIN = 256
WSH = 8
KB = 128
EBLK = E // 16


def _rsqrt_nr(s):
    i = lax.bitcast_convert_type(s, jnp.int32)
    i = jnp.int32(0x5F3759DF) - (i >> 1)
    r = lax.bitcast_convert_type(i, jnp.float32)
    for _ in range(3):
        r = r * (1.5 - 0.5 * s * r * r)
    return r


def _sc_body(rx_h, ry_h, rz_h, qx_h, qy_h, qz_h, er_h, eq_h, eb_h, feat_h,
             zero_h, qf_h, ws_h,
             rx_v, ry_v, rz_v, qx_v, qy_v, qz_v,
             acc_v, wsw_v, feat_a, feat_b, er_a, eq_a, er_b, eq_b,
             d_a, qc_a, d_b, qc_b, eb_v,
             sem_ga, sem_gb, sem_f, sem_w, sem_b, sem_ia, sem_ib):
    wid = lax.axis_index("c") * NS + lax.axis_index("s")
    row_lo = wid * RW
    row_hi = jnp.minimum(row_lo + RW, N_Q)

    pltpu.async_copy(eb_h, eb_v, sem_ga)
    row_lo8 = pl.multiple_of(row_lo, 64)
    pltpu.async_copy(rx_h, rx_v, sem_b)
    pltpu.async_copy(ry_h, ry_v, sem_b)
    pltpu.async_copy(rz_h, rz_v, sem_b)
    pltpu.async_copy(qx_h.at[pl.ds(row_lo8, RW)], qx_v, sem_b)
    pltpu.async_copy(qy_h.at[pl.ds(row_lo8, RW)], qy_v, sem_b)
    pltpu.async_copy(qz_h.at[pl.ds(row_lo8, RW)], qz_v, sem_b)
    pltpu.async_copy(zero_h, acc_v, sem_b)
    pltpu.async_copy(zero_h.at[pl.ds(0, WIN)], wsw_v, sem_b)

    pltpu.make_async_copy(eb_h, eb_v, sem_ga).wait()
    widv = jnp.full((16,), wid, jnp.int32)
    e_lo = plsc.load_gather(eb_v, [widv])[0]
    e_hi = plsc.load_gather(eb_v, [widv + 1])[0]
    a_lo = e_lo & jnp.int32(-8)

    pltpu.make_async_copy(rx_h, rx_v, sem_b).wait()
    pltpu.make_async_copy(ry_h, ry_v, sem_b).wait()
    pltpu.make_async_copy(rz_h, rz_v, sem_b).wait()
    pltpu.make_async_copy(qx_h.at[pl.ds(row_lo8, RW)], qx_v, sem_b).wait()
    pltpu.make_async_copy(qy_h.at[pl.ds(row_lo8, RW)], qy_v, sem_b).wait()
    pltpu.make_async_copy(qz_h.at[pl.ds(row_lo8, RW)], qz_v, sem_b).wait()
    pltpu.make_async_copy(zero_h, acc_v, sem_b).wait()
    pltpu.make_async_copy(zero_h.at[pl.ds(0, WIN)], wsw_v, sem_b).wait()

    def flush(wb, nrows):
        n32 = nrows >> 5

        def issue(t, _):
            o1 = pl.multiple_of((wb + t * 32) * D, 4096)
            o2 = pl.multiple_of(wb + t * 32, 32)
            pltpu.async_copy(acc_v.at[pl.ds(t * 4096, 4096)],
                             qf_h.at[pl.ds(o1, 4096)], sem_f)
            pltpu.async_copy(wsw_v.at[pl.ds(t * 32, 32)],
                             ws_h.at[pl.ds(o2, 32)], sem_w)
            return 0

        def drain(t, _):
            o1 = pl.multiple_of((wb + t * 32) * D, 4096)
            o2 = pl.multiple_of(wb + t * 32, 32)
            pltpu.make_async_copy(acc_v.at[pl.ds(t * 4096, 4096)],
                                  qf_h.at[pl.ds(o1, 4096)],
                                  sem_f).wait()
            pltpu.make_async_copy(wsw_v.at[pl.ds(t * 32, 32)],
                                  ws_h.at[pl.ds(o2, 32)],
                                  sem_w).wait()
            return 0

        lax.fori_loop(0, n32, issue, 0)
        lax.fori_loop(0, n32, drain, 0)
        pltpu.async_copy(zero_h, acc_v, sem_b)
        pltpu.async_copy(zero_h.at[pl.ds(0, WIN)], wsw_v, sem_b)
        pltpu.make_async_copy(zero_h, acc_v, sem_b).wait()
        pltpu.make_async_copy(zero_h.at[pl.ds(0, WIN)], wsw_v, sem_b).wait()

    def advance(wb):
        flush(wb, jnp.minimum(WIN, row_hi - wb))
        return wb + WIN

    lane = lax.iota(jnp.int32, 16)
    lane0 = lane == 0

    def groups(feat_v, d_v, qc_v, wb):
        def group_body(g, wb):
            eqb = qc_v[pl.ds(16 * g, 16)]
            db = d_v[pl.ds(16 * g, 16)]
            q0 = eqb[0]
            q15 = eqb[15]
            wb = lax.fori_loop(0, (q0 - wb) >> WSH,
                               lambda _, w: advance(w), wb)

            def fast(wb):
                for k in range(16):
                    ro = eqb[k] - wb
                    dv = jnp.full((16,), db[k])
                    for j in range(D // 16):
                        f = feat_v[16 * g + k, pl.ds(16 * j, 16)]
                        plsc.addupdate(acc_v.at[pl.ds(ro * D + 16 * j, 16)],
                                       f * dv)
                    plsc.addupdate_scatter(wsw_v,
                                           [jnp.full((16,), ro, jnp.int32)],
                                           db, mask=lane == k)
                return wb

            def slow(wb):
                def eb(k, w):
                    iv = jnp.full((16,), 16 * g + k, jnp.int32)
                    q = plsc.load_gather(qc_v, [iv])[0]
                    dbv = plsc.load_gather(d_v, [iv])
                    w = lax.fori_loop(0, (q - w) >> WSH,
                                      lambda _, x: advance(x), w)
                    ro = q - w
                    for j in range(D // 16):
                        f = feat_v[16 * g + k, pl.ds(16 * j, 16)]
                        plsc.addupdate(acc_v.at[pl.ds(ro * D + 16 * j, 16)],
                                       f * dbv)
                    plsc.addupdate_scatter(wsw_v,
                                           [jnp.full((16,), ro, jnp.int32)],
                                           dbv, mask=lane0)
                    return w

                return lax.fori_loop(0, 16, eb, wb)

            return lax.cond(q15 - wb < WIN, fast, slow, wb)

        return lax.fori_loop(0, KB // 16, group_body, wb)

    def issue_idx(b, er_v, eq_v, sem):
        base = pl.multiple_of(a_lo + b * KB, 8)
        pltpu.async_copy(er_h.at[pl.ds(base, KB)], er_v, sem)
        pltpu.async_copy(eq_h.at[pl.ds(base, KB)], eq_v, sem)

    def wait_idx(b, er_v, eq_v, sem):
        base = pl.multiple_of(a_lo + b * KB, 8)
        pltpu.make_async_copy(er_h.at[pl.ds(base, KB)], er_v, sem).wait()
        pltpu.make_async_copy(eq_h.at[pl.ds(base, KB)], eq_v, sem).wait()

    def weights(er_v, eq_v, d_v, qc_v):
        for g in range(KB // 16):
            qi = eq_v[pl.ds(16 * g, 16)]
            ri = er_v[pl.ds(16 * g, 16)]
            valid = (qi >= row_lo) & (qi < row_hi)
            qidx = jnp.where(valid, qi - row_lo, 0)
            qx = plsc.load_gather(qx_v, [qidx], mask=valid)
            qy = plsc.load_gather(qy_v, [qidx], mask=valid)
            qz = plsc.load_gather(qz_v, [qidx], mask=valid)
            rxx = plsc.load_gather(rx_v, [ri])
            ryy = plsc.load_gather(ry_v, [ri])
            rzz = plsc.load_gather(rz_v, [ri])
            dx = rxx - qx
            dy = ryy - qy
            dz = rzz - qz
            s = jnp.maximum(dx * dx + dy * dy + dz * dz, 1e-30)
            r = _rsqrt_nr(s)
            dd = r / (1.0 + 1e-8 * r)
            d_v[pl.ds(16 * g, 16)] = jnp.where(valid, dd, 0.0)
            qc_v[pl.ds(16 * g, 16)] = jnp.minimum(
                jnp.maximum(qi, row_lo), row_hi - 1)

    nblk = (e_hi - a_lo + (KB - 1)) >> 7
    npair = jnp.maximum((nblk + 1) >> 1, 1)
    issue_idx(0, er_a, eq_a, sem_ia)
    wait_idx(0, er_a, eq_a, sem_ia)
    pltpu.async_copy(feat_h.at[er_a], feat_a, sem_ga)
    issue_idx(1, er_b, eq_b, sem_ib)

    def pair_body(i, wb):
        wait_idx(2 * i + 1, er_b, eq_b, sem_ib)
        pltpu.async_copy(feat_h.at[er_b], feat_b, sem_gb)
        weights(er_a, eq_a, d_a, qc_a)
        pltpu.make_async_copy(feat_h.at[er_a], feat_a, sem_ga).wait()
        issue_idx(2 * i + 2, er_a, eq_a, sem_ia)
        wb = groups(feat_a, d_a, qc_a, wb)
        wait_idx(2 * i + 2, er_a, eq_a, sem_ia)
        pltpu.async_copy(feat_h.at[er_a], feat_a, sem_ga)
        weights(er_b, eq_b, d_b, qc_b)
        pltpu.make_async_copy(feat_h.at[er_b], feat_b, sem_gb).wait()
        issue_idx(2 * i + 3, er_b, eq_b, sem_ib)
        wb = groups(feat_b, d_b, qc_b, wb)
        return wb

    wb = lax.fori_loop(0, npair, pair_body, row_lo)
    pltpu.make_async_copy(feat_h.at[er_a], feat_a, sem_ga).wait()
    wait_idx(2 * npair + 1, er_b, eq_b, sem_ib)
    lax.fori_loop(0, (row_hi - wb + (WIN - 1)) >> WSH,
                  lambda _, w: advance(w), wb)


def _sc_edge_conv(ref_bxyz, query_bxyz, e_ref, e_query, feat):
    mesh = plsc.VectorSubcoreMesh(core_axis_name="c", subcore_axis_name="s")
    rx = ref_bxyz[:, 1] + 0.0
    ry = ref_bxyz[:, 2] + 0.0
    rz = ref_bxyz[:, 3] + 0.0
    qpad = jnp.pad(query_bxyz[:, 1:4], ((0, NQ_PAD - N_Q), (0, 0)))
    qx = qpad[:, 0] + 0.0
    qy = qpad[:, 1] + 0.0
    qz = qpad[:, 2] + 0.0
    er_p = jnp.pad(e_ref, (0, 6 * KB))
    eq_p = jnp.pad(e_query, (0, 6 * KB), constant_values=N_Q)
    bnd = jnp.minimum(jnp.arange(NW + 1, dtype=jnp.int32) * RW, N_Q)
    ebnd = jnp.searchsorted(e_query, bnd).astype(jnp.int32)
    ebnd = jnp.pad(ebnd, (0, 48 - (NW + 1)))
    zeros = jnp.zeros((WIN * D,), jnp.float32)
    run = pl.kernel(
        _sc_body,
        out_type=(jax.ShapeDtypeStruct((N_Q * D,), jnp.float32),
                  jax.ShapeDtypeStruct((N_Q,), jnp.float32)),
        mesh=mesh,
        compiler_params=pltpu.CompilerParams(needs_layout_passes=False),
        scratch_types=[
            pltpu.VMEM((N_REF,), jnp.float32),
            pltpu.VMEM((N_REF,), jnp.float32),
            pltpu.VMEM((N_REF,), jnp.float32),
            pltpu.VMEM((RW,), jnp.float32),
            pltpu.VMEM((RW,), jnp.float32),
            pltpu.VMEM((RW,), jnp.float32),
            pltpu.VMEM((WIN * D,), jnp.float32),
            pltpu.VMEM((WIN,), jnp.float32),
            pltpu.VMEM((KB, D), jnp.float32),
            pltpu.VMEM((KB, D), jnp.float32),
            pltpu.VMEM((KB,), jnp.int32),
            pltpu.VMEM((KB,), jnp.int32),
            pltpu.VMEM((KB,), jnp.int32),
            pltpu.VMEM((KB,), jnp.int32),
            pltpu.VMEM((KB,), jnp.float32),
            pltpu.VMEM((KB,), jnp.int32),
            pltpu.VMEM((KB,), jnp.float32),
            pltpu.VMEM((KB,), jnp.int32),
            pltpu.VMEM((48,), jnp.int32),
            pltpu.SemaphoreType.DMA,
            pltpu.SemaphoreType.DMA,
            pltpu.SemaphoreType.DMA,
            pltpu.SemaphoreType.DMA,
            pltpu.SemaphoreType.DMA,
            pltpu.SemaphoreType.DMA,
            pltpu.SemaphoreType.DMA,
        ],
    )
    qf_flat, wsum = run(rx, ry, rz, qx, qy, qz, er_p, eq_p, ebnd, feat, zeros)
    return qf_flat.reshape(N_Q, D), wsum


def _k1_body(x_ref, wt_ref, g_ref, b_ref, o_ref):
    y = jnp.dot(x_ref[...], wt_ref[...], preferred_element_type=jnp.float32)
    m = jnp.mean(y, axis=0, keepdims=True)
    v = jnp.mean((y - m) ** 2, axis=0, keepdims=True)
    o_ref[...] = g_ref[...] * (y - m) / jnp.sqrt(v + 1e-5) + b_ref[...]


def _ref_branch(ref_feat, W_f0, gamma, beta):
    return pl.pallas_call(
        _k1_body,
        out_shape=jax.ShapeDtypeStruct((N_REF, D), jnp.float32),
    )(ref_feat, W_f0.T, gamma.reshape(1, D), beta.reshape(1, D))


_BQ = 1000
_NBQ = N_Q // _BQ


def _stats_body(x_ref, wt_ref, ssum_ref, ssq_ref, acc_ref):
    i = pl.program_id(0)

    @pl.when(i == 0)
    def _():
        acc_ref[...] = jnp.zeros_like(acc_ref)

    y = jnp.dot(x_ref[...], wt_ref[...], preferred_element_type=jnp.float32)
    acc_ref[0:1, :] += jnp.sum(y, axis=0, keepdims=True)
    acc_ref[1:2, :] += jnp.sum(y * y, axis=0, keepdims=True)

    @pl.when(i == _NBQ - 1)
    def _():
        ssum_ref[...] = acc_ref[0:1, :]
        ssq_ref[...] = acc_ref[1:2, :]


def _skip_stats(query_feat, W_s0):
    return pl.pallas_call(
        _stats_body,
        grid=(_NBQ,),
        in_specs=[
            pl.BlockSpec((_BQ, D), lambda i: (i, 0)),
            pl.BlockSpec((D, D), lambda i: (0, 0)),
        ],
        out_specs=[
            pl.BlockSpec((1, D), lambda i: (0, 0)),
            pl.BlockSpec((1, D), lambda i: (0, 0)),
        ],
        out_shape=[
            jax.ShapeDtypeStruct((1, D), jnp.float32),
            jax.ShapeDtypeStruct((1, D), jnp.float32),
        ],
        scratch_shapes=[pltpu.VMEM((2, D), jnp.float32)],
    )(query_feat, W_s0.T)


def _mid_body(x_ref, wt_ref, sc_ref, bi_ref, acc_ref, w_ref, w1t_ref, b1_ref,
              z_ref, zsum_ref, zsq_ref, st_ref):
    i = pl.program_id(0)

    @pl.when(i == 0)
    def _():
        st_ref[...] = jnp.zeros_like(st_ref)

    ys = jnp.dot(x_ref[...], wt_ref[...], preferred_element_type=jnp.float32)
    skip = ys * sc_ref[...] + bi_ref[...]
    w = w_ref[...].reshape(_BQ, 1)
    inv = jnp.where(w > 0, 1.0 / w, 0.0)
    h = jnp.maximum(acc_ref[...] * inv + skip, 0.0)
    z = jnp.dot(h, w1t_ref[...], preferred_element_type=jnp.float32) + b1_ref[...]
    z_ref[...] = z
    st_ref[0:1, :] += jnp.sum(z, axis=0, keepdims=True)
    st_ref[1:2, :] += jnp.sum(z * z, axis=0, keepdims=True)

    @pl.when(i == _NBQ - 1)
    def _():
        zsum_ref[...] = st_ref[0:1, :]
        zsq_ref[...] = st_ref[1:2, :]


def _mid(query_feat, W_s0, scale_s, bias_s, qf_acc, wsum, W1, b1):
    return pl.pallas_call(
        _mid_body,
        grid=(_NBQ,),
        in_specs=[
            pl.BlockSpec((_BQ, D), lambda i: (i, 0)),
            pl.BlockSpec((D, D), lambda i: (0, 0)),
            pl.BlockSpec((1, D), lambda i: (0, 0)),
            pl.BlockSpec((1, D), lambda i: (0, 0)),
            pl.BlockSpec((_BQ, D), lambda i: (i, 0)),
            pl.BlockSpec((1, 1, _BQ), lambda i: (i, 0, 0)),
            pl.BlockSpec((D, D), lambda i: (0, 0)),
            pl.BlockSpec((1, D), lambda i: (0, 0)),
        ],
        out_specs=[
            pl.BlockSpec((_BQ, D), lambda i: (i, 0)),
            pl.BlockSpec((1, D), lambda i: (0, 0)),
            pl.BlockSpec((1, D), lambda i: (0, 0)),
        ],
        out_shape=[
            jax.ShapeDtypeStruct((N_Q, D), jnp.float32),
            jax.ShapeDtypeStruct((1, D), jnp.float32),
            jax.ShapeDtypeStruct((1, D), jnp.float32),
        ],
        scratch_shapes=[pltpu.VMEM((2, D), jnp.float32)],
    )(query_feat, W_s0.T, scale_s, bias_s, qf_acc,
      wsum.reshape(_NBQ, 1, _BQ), W1.T, b1.reshape(1, D))


def _fin_body(z_ref, sc_ref, bi_ref, o_ref):
    o_ref[...] = jnp.maximum(z_ref[...] * sc_ref[...] + bi_ref[...], 0.0)


def _final(z, scale_z, bias_z):
    return pl.pallas_call(
        _fin_body,
        grid=(_NBQ,),
        in_specs=[
            pl.BlockSpec((_BQ, D), lambda i: (i, 0)),
            pl.BlockSpec((1, D), lambda i: (0, 0)),
            pl.BlockSpec((1, D), lambda i: (0, 0)),
        ],
        out_specs=pl.BlockSpec((_BQ, D), lambda i: (i, 0)),
        out_shape=jax.ShapeDtypeStruct((N_Q, D), jnp.float32),
    )(z, scale_z, bias_z)


def kernel(ref_bxyz, query_bxyz, ref_feat, query_feat, e_ref, e_query,
           W_f0, gamma_f0, beta_f0, W_s0, gamma_s0, beta_s0,
           W1, b1, gamma1, beta1):
    feat2 = _ref_branch(ref_feat, W_f0, gamma_f0, beta_f0)
    qf_acc, wsum = _sc_edge_conv(ref_bxyz, query_bxyz, e_ref, e_query, feat2)
    ssum, ssq = _skip_stats(query_feat, W_s0)
    n = jnp.float32(N_Q)
    m_s = ssum / n
    v_s = ssq / n - m_s * m_s
    scale_s = gamma_s0.reshape(1, D) / jnp.sqrt(v_s + 1e-5)
    bias_s = beta_s0.reshape(1, D) - m_s * scale_s
    z, zsum, zsq = _mid(query_feat, W_s0, scale_s, bias_s, qf_acc, wsum, W1, b1)
    m_z = zsum / n
    v_z = zsq / n - m_z * m_z
    scale_z = gamma1.reshape(1, D) / jnp.sqrt(v_z + 1e-5)
    bias_z = beta1.reshape(1, D) - m_z * scale_z
    return _final(z, scale_z, bias_z)


# speedup vs baseline: 26.1322x; 1.0163x over previous
import functools

import jax
import jax.numpy as jnp
from jax import lax
from jax.experimental import pallas as pl
from jax.experimental.pallas import tpu as pltpu
from jax.experimental.pallas import tpu_sc as plsc

N_REF = 10000
N_Q = 100000
E = 300000
D = 128

NC, NS = 2, 16
NW = NC * NS
RW = 3136
NQ_PAD = NW * RW
WIN = 256
WSH = 8
KB = 128
EBLK = E // 16


def _rsqrt_nr(s):
    i = lax.bitcast_convert_type(s, jnp.int32)
    i = jnp.int32(0x5F3759DF) - (i >> 1)
    r = lax.bitcast_convert_type(i, jnp.float32)
    for _ in range(3):
        r = r * (1.5 - 0.5 * s * r * r)
    return r


def _sc_body(rx_h, ry_h, rz_h, qx_h, qy_h, qz_h, er_h, eq_h, eb_h, feat_h,
             zero_h, qf_h, ws_h,
             rx_v, ry_v, rz_v, qx_v, qy_v, qz_v,
             acc_v, wsw_v, feat_a, feat_b, er_a, eq_a, er_b, eq_b,
             d_a, qc_a, d_b, qc_b, eb_v,
             sem_ga, sem_gb, sem_f, sem_w, sem_b, sem_ia, sem_ib):
    wid = lax.axis_index("c") * NS + lax.axis_index("s")
    row_lo = wid * RW
    row_hi = jnp.minimum(row_lo + RW, N_Q)

    pltpu.async_copy(eb_h, eb_v, sem_ga)
    row_lo8 = pl.multiple_of(row_lo, 64)
    pltpu.async_copy(rx_h, rx_v, sem_b)
    pltpu.async_copy(ry_h, ry_v, sem_b)
    pltpu.async_copy(rz_h, rz_v, sem_b)
    pltpu.async_copy(qx_h.at[pl.ds(row_lo8, RW)], qx_v, sem_b)
    pltpu.async_copy(qy_h.at[pl.ds(row_lo8, RW)], qy_v, sem_b)
    pltpu.async_copy(qz_h.at[pl.ds(row_lo8, RW)], qz_v, sem_b)
    pltpu.async_copy(zero_h, acc_v, sem_b)
    pltpu.async_copy(zero_h.at[pl.ds(0, WIN)], wsw_v, sem_b)

    pltpu.make_async_copy(eb_h, eb_v, sem_ga).wait()
    widv = jnp.full((16,), wid, jnp.int32)
    e_lo = plsc.load_gather(eb_v, [widv])[0]
    e_hi = plsc.load_gather(eb_v, [widv + 1])[0]
    a_lo = e_lo & jnp.int32(-8)

    pltpu.make_async_copy(rx_h, rx_v, sem_b).wait()
    pltpu.make_async_copy(ry_h, ry_v, sem_b).wait()
    pltpu.make_async_copy(rz_h, rz_v, sem_b).wait()
    pltpu.make_async_copy(qx_h.at[pl.ds(row_lo8, RW)], qx_v, sem_b).wait()
    pltpu.make_async_copy(qy_h.at[pl.ds(row_lo8, RW)], qy_v, sem_b).wait()
    pltpu.make_async_copy(qz_h.at[pl.ds(row_lo8, RW)], qz_v, sem_b).wait()
    pltpu.make_async_copy(zero_h, acc_v, sem_b).wait()
    pltpu.make_async_copy(zero_h.at[pl.ds(0, WIN)], wsw_v, sem_b).wait()

    def flush(wb, nrows):
        n32 = nrows >> 5

        def issue(t, _):
            o1 = pl.multiple_of((wb + t * 32) * D, 4096)
            o2 = pl.multiple_of(wb + t * 32, 32)
            pltpu.async_copy(acc_v.at[pl.ds(t * 4096, 4096)],
                             qf_h.at[pl.ds(o1, 4096)], sem_f)
            pltpu.async_copy(wsw_v.at[pl.ds(t * 32, 32)],
                             ws_h.at[pl.ds(o2, 32)], sem_w)
            return 0

        def drain(t, _):
            o1 = pl.multiple_of((wb + t * 32) * D, 4096)
            o2 = pl.multiple_of(wb + t * 32, 32)
            pltpu.make_async_copy(acc_v.at[pl.ds(t * 4096, 4096)],
                                  qf_h.at[pl.ds(o1, 4096)],
                                  sem_f).wait()
            pltpu.make_async_copy(wsw_v.at[pl.ds(t * 32, 32)],
                                  ws_h.at[pl.ds(o2, 32)],
                                  sem_w).wait()
            pltpu.async_copy(zero_h.at[pl.ds(t * 4096, 4096)],
                             acc_v.at[pl.ds(t * 4096, 4096)], sem_b)
            pltpu.async_copy(zero_h.at[pl.ds(t * 32, 32)],
                             wsw_v.at[pl.ds(t * 32, 32)], sem_b)
            return 0

        def zwait(t, _):
            pltpu.make_async_copy(zero_h.at[pl.ds(t * 4096, 4096)],
                                  acc_v.at[pl.ds(t * 4096, 4096)],
                                  sem_b).wait()
            pltpu.make_async_copy(zero_h.at[pl.ds(t * 32, 32)],
                                  wsw_v.at[pl.ds(t * 32, 32)],
                                  sem_b).wait()
            return 0

        lax.fori_loop(0, n32, issue, 0)
        lax.fori_loop(0, n32, drain, 0)
        lax.fori_loop(0, n32, zwait, 0)

    def advance(wb):
        flush(wb, jnp.minimum(WIN, row_hi - wb))
        return wb + WIN

    lane = lax.iota(jnp.int32, 16)
    lane0 = lane == 0

    def groups(feat_v, d_v, qc_v, wb):
        def group_body(g, wb):
            eqb = qc_v[pl.ds(16 * g, 16)]
            db = d_v[pl.ds(16 * g, 16)]
            q0 = eqb[0]
            q15 = eqb[15]
            wb = lax.fori_loop(0, (q0 - wb) >> WSH,
                               lambda _, w: advance(w), wb)

            def fast(wb):
                for k in range(16):
                    ro = eqb[k] - wb
                    dv = jnp.full((16,), db[k])
                    for j in range(D // 16):
                        f = feat_v[16 * g + k, pl.ds(16 * j, 16)]
                        plsc.addupdate(acc_v.at[pl.ds(ro * D + 16 * j, 16)],
                                       f * dv)
                    plsc.addupdate_scatter(wsw_v,
                                           [jnp.full((16,), ro, jnp.int32)],
                                           db, mask=lane == k)
                return wb

            def slow(wb):
                def eb(k, w):
                    iv = jnp.full((16,), 16 * g + k, jnp.int32)
                    q = plsc.load_gather(qc_v, [iv])[0]
                    dbv = plsc.load_gather(d_v, [iv])
                    w = lax.fori_loop(0, (q - w) >> WSH,
                                      lambda _, x: advance(x), w)
                    ro = q - w
                    for j in range(D // 16):
                        f = feat_v[16 * g + k, pl.ds(16 * j, 16)]
                        plsc.addupdate(acc_v.at[pl.ds(ro * D + 16 * j, 16)],
                                       f * dbv)
                    plsc.addupdate_scatter(wsw_v,
                                           [jnp.full((16,), ro, jnp.int32)],
                                           dbv, mask=lane0)
                    return w

                return lax.fori_loop(0, 16, eb, wb)

            return lax.cond(q15 - wb < WIN, fast, slow, wb)

        return lax.fori_loop(0, KB // 16, group_body, wb)

    def issue_idx(b, er_v, eq_v, sem):
        base = pl.multiple_of(a_lo + b * KB, 8)
        pltpu.async_copy(er_h.at[pl.ds(base, KB)], er_v, sem)
        pltpu.async_copy(eq_h.at[pl.ds(base, KB)], eq_v, sem)

    def wait_idx(b, er_v, eq_v, sem):
        base = pl.multiple_of(a_lo + b * KB, 8)
        pltpu.make_async_copy(er_h.at[pl.ds(base, KB)], er_v, sem).wait()
        pltpu.make_async_copy(eq_h.at[pl.ds(base, KB)], eq_v, sem).wait()

    def weights(er_v, eq_v, d_v, qc_v):
        for g in range(KB // 16):
            qi = eq_v[pl.ds(16 * g, 16)]
            ri = er_v[pl.ds(16 * g, 16)]
            valid = (qi >= row_lo) & (qi < row_hi)
            qidx = jnp.where(valid, qi - row_lo, 0)
            qx = plsc.load_gather(qx_v, [qidx], mask=valid)
            qy = plsc.load_gather(qy_v, [qidx], mask=valid)
            qz = plsc.load_gather(qz_v, [qidx], mask=valid)
            rxx = plsc.load_gather(rx_v, [ri])
            ryy = plsc.load_gather(ry_v, [ri])
            rzz = plsc.load_gather(rz_v, [ri])
            dx = rxx - qx
            dy = ryy - qy
            dz = rzz - qz
            s = jnp.maximum(dx * dx + dy * dy + dz * dz, 1e-30)
            r = _rsqrt_nr(s)
            dd = r / (1.0 + 1e-8 * r)
            d_v[pl.ds(16 * g, 16)] = jnp.where(valid, dd, 0.0)
            qc_v[pl.ds(16 * g, 16)] = jnp.minimum(
                jnp.maximum(qi, row_lo), row_hi - 1)

    nblk = (e_hi - a_lo + (KB - 1)) >> 7
    npair = jnp.maximum((nblk + 1) >> 1, 1)
    issue_idx(0, er_a, eq_a, sem_ia)
    wait_idx(0, er_a, eq_a, sem_ia)
    pltpu.async_copy(feat_h.at[er_a], feat_a, sem_ga)
    issue_idx(1, er_b, eq_b, sem_ib)

    def pair_body(i, wb):
        wait_idx(2 * i + 1, er_b, eq_b, sem_ib)
        pltpu.async_copy(feat_h.at[er_b], feat_b, sem_gb)
        weights(er_a, eq_a, d_a, qc_a)
        pltpu.make_async_copy(feat_h.at[er_a], feat_a, sem_ga).wait()
        issue_idx(2 * i + 2, er_a, eq_a, sem_ia)
        wb = groups(feat_a, d_a, qc_a, wb)
        wait_idx(2 * i + 2, er_a, eq_a, sem_ia)
        pltpu.async_copy(feat_h.at[er_a], feat_a, sem_ga)
        weights(er_b, eq_b, d_b, qc_b)
        pltpu.make_async_copy(feat_h.at[er_b], feat_b, sem_gb).wait()
        issue_idx(2 * i + 3, er_b, eq_b, sem_ib)
        wb = groups(feat_b, d_b, qc_b, wb)
        return wb

    wb = lax.fori_loop(0, npair, pair_body, row_lo)
    pltpu.make_async_copy(feat_h.at[er_a], feat_a, sem_ga).wait()
    wait_idx(2 * npair + 1, er_b, eq_b, sem_ib)
    lax.fori_loop(0, (row_hi - wb + (WIN - 1)) >> WSH,
                  lambda _, w: advance(w), wb)


def _sc_edge_conv(ref_bxyz, query_bxyz, e_ref, e_query, feat):
    mesh = plsc.VectorSubcoreMesh(core_axis_name="c", subcore_axis_name="s")
    rx = ref_bxyz[:, 1] + 0.0
    ry = ref_bxyz[:, 2] + 0.0
    rz = ref_bxyz[:, 3] + 0.0
    qpad = jnp.pad(query_bxyz[:, 1:4], ((0, NQ_PAD - N_Q), (0, 0)))
    qx = qpad[:, 0] + 0.0
    qy = qpad[:, 1] + 0.0
    qz = qpad[:, 2] + 0.0
    er_p = jnp.pad(e_ref, (0, 6 * KB))
    eq_p = jnp.pad(e_query, (0, 6 * KB), constant_values=N_Q)
    bnd = jnp.minimum(jnp.arange(NW + 1, dtype=jnp.int32) * RW, N_Q)
    ebnd = jnp.searchsorted(e_query, bnd).astype(jnp.int32)
    ebnd = jnp.pad(ebnd, (0, 48 - (NW + 1)))
    zeros = jnp.zeros((WIN * D,), jnp.float32)
    run = pl.kernel(
        _sc_body,
        out_type=(jax.ShapeDtypeStruct((N_Q * D,), jnp.float32),
                  jax.ShapeDtypeStruct((N_Q,), jnp.float32)),
        mesh=mesh,
        compiler_params=pltpu.CompilerParams(needs_layout_passes=False),
        scratch_types=[
            pltpu.VMEM((N_REF,), jnp.float32),
            pltpu.VMEM((N_REF,), jnp.float32),
            pltpu.VMEM((N_REF,), jnp.float32),
            pltpu.VMEM((RW,), jnp.float32),
            pltpu.VMEM((RW,), jnp.float32),
            pltpu.VMEM((RW,), jnp.float32),
            pltpu.VMEM((WIN * D,), jnp.float32),
            pltpu.VMEM((WIN,), jnp.float32),
            pltpu.VMEM((KB, D), jnp.float32),
            pltpu.VMEM((KB, D), jnp.float32),
            pltpu.VMEM((KB,), jnp.int32),
            pltpu.VMEM((KB,), jnp.int32),
            pltpu.VMEM((KB,), jnp.int32),
            pltpu.VMEM((KB,), jnp.int32),
            pltpu.VMEM((KB,), jnp.float32),
            pltpu.VMEM((KB,), jnp.int32),
            pltpu.VMEM((KB,), jnp.float32),
            pltpu.VMEM((KB,), jnp.int32),
            pltpu.VMEM((48,), jnp.int32),
            pltpu.SemaphoreType.DMA,
            pltpu.SemaphoreType.DMA,
            pltpu.SemaphoreType.DMA,
            pltpu.SemaphoreType.DMA,
            pltpu.SemaphoreType.DMA,
            pltpu.SemaphoreType.DMA,
            pltpu.SemaphoreType.DMA,
        ],
    )
    qf_flat, wsum = run(rx, ry, rz, qx, qy, qz, er_p, eq_p, ebnd, feat, zeros)
    return qf_flat.reshape(N_Q, D), wsum


def _k1_body(x_ref, wt_ref, g_ref, b_ref, o_ref):
    y = jnp.dot(x_ref[...], wt_ref[...], preferred_element_type=jnp.float32)
    m = jnp.mean(y, axis=0, keepdims=True)
    v = jnp.mean((y - m) ** 2, axis=0, keepdims=True)
    o_ref[...] = g_ref[...] * (y - m) / jnp.sqrt(v + 1e-5) + b_ref[...]


def _ref_branch(ref_feat, W_f0, gamma, beta):
    return pl.pallas_call(
        _k1_body,
        out_shape=jax.ShapeDtypeStruct((N_REF, D), jnp.float32),
    )(ref_feat, W_f0.T, gamma.reshape(1, D), beta.reshape(1, D))


_BQ = 1000
_NBQ = N_Q // _BQ


def _stats_body(x_ref, wt_ref, ssum_ref, ssq_ref, acc_ref):
    i = pl.program_id(0)

    @pl.when(i == 0)
    def _():
        acc_ref[...] = jnp.zeros_like(acc_ref)

    y = jnp.dot(x_ref[...], wt_ref[...], preferred_element_type=jnp.float32)
    acc_ref[0:1, :] += jnp.sum(y, axis=0, keepdims=True)
    acc_ref[1:2, :] += jnp.sum(y * y, axis=0, keepdims=True)

    @pl.when(i == _NBQ - 1)
    def _():
        ssum_ref[...] = acc_ref[0:1, :]
        ssq_ref[...] = acc_ref[1:2, :]


def _skip_stats(query_feat, W_s0):
    return pl.pallas_call(
        _stats_body,
        grid=(_NBQ,),
        in_specs=[
            pl.BlockSpec((_BQ, D), lambda i: (i, 0)),
            pl.BlockSpec((D, D), lambda i: (0, 0)),
        ],
        out_specs=[
            pl.BlockSpec((1, D), lambda i: (0, 0)),
            pl.BlockSpec((1, D), lambda i: (0, 0)),
        ],
        out_shape=[
            jax.ShapeDtypeStruct((1, D), jnp.float32),
            jax.ShapeDtypeStruct((1, D), jnp.float32),
        ],
        scratch_shapes=[pltpu.VMEM((2, D), jnp.float32)],
    )(query_feat, W_s0.T)


def _mid_body(x_ref, wt_ref, sc_ref, bi_ref, acc_ref, w_ref, w1t_ref, b1_ref,
              z_ref, zsum_ref, zsq_ref, st_ref):
    i = pl.program_id(0)

    @pl.when(i == 0)
    def _():
        st_ref[...] = jnp.zeros_like(st_ref)

    ys = jnp.dot(x_ref[...], wt_ref[...], preferred_element_type=jnp.float32)
    skip = ys * sc_ref[...] + bi_ref[...]
    w = w_ref[...].reshape(_BQ, 1)
    inv = jnp.where(w > 0, 1.0 / w, 0.0)
    h = jnp.maximum(acc_ref[...] * inv + skip, 0.0)
    z = jnp.dot(h, w1t_ref[...], preferred_element_type=jnp.float32) + b1_ref[...]
    z_ref[...] = z
    st_ref[0:1, :] += jnp.sum(z, axis=0, keepdims=True)
    st_ref[1:2, :] += jnp.sum(z * z, axis=0, keepdims=True)

    @pl.when(i == _NBQ - 1)
    def _():
        zsum_ref[...] = st_ref[0:1, :]
        zsq_ref[...] = st_ref[1:2, :]


def _mid(query_feat, W_s0, scale_s, bias_s, qf_acc, wsum, W1, b1):
    return pl.pallas_call(
        _mid_body,
        grid=(_NBQ,),
        in_specs=[
            pl.BlockSpec((_BQ, D), lambda i: (i, 0)),
            pl.BlockSpec((D, D), lambda i: (0, 0)),
            pl.BlockSpec((1, D), lambda i: (0, 0)),
            pl.BlockSpec((1, D), lambda i: (0, 0)),
            pl.BlockSpec((_BQ, D), lambda i: (i, 0)),
            pl.BlockSpec((1, 1, _BQ), lambda i: (i, 0, 0)),
            pl.BlockSpec((D, D), lambda i: (0, 0)),
            pl.BlockSpec((1, D), lambda i: (0, 0)),
        ],
        out_specs=[
            pl.BlockSpec((_BQ, D), lambda i: (i, 0)),
            pl.BlockSpec((1, D), lambda i: (0, 0)),
            pl.BlockSpec((1, D), lambda i: (0, 0)),
        ],
        out_shape=[
            jax.ShapeDtypeStruct((N_Q, D), jnp.float32),
            jax.ShapeDtypeStruct((1, D), jnp.float32),
            jax.ShapeDtypeStruct((1, D), jnp.float32),
        ],
        scratch_shapes=[pltpu.VMEM((2, D), jnp.float32)],
    )(query_feat, W_s0.T, scale_s, bias_s, qf_acc,
      wsum.reshape(_NBQ, 1, _BQ), W1.T, b1.reshape(1, D))


def _fin_body(z_ref, sc_ref, bi_ref, o_ref):
    o_ref[...] = jnp.maximum(z_ref[...] * sc_ref[...] + bi_ref[...], 0.0)


def _final(z, scale_z, bias_z):
    return pl.pallas_call(
        _fin_body,
        grid=(_NBQ,),
        in_specs=[
            pl.BlockSpec((_BQ, D), lambda i: (i, 0)),
            pl.BlockSpec((1, D), lambda i: (0, 0)),
            pl.BlockSpec((1, D), lambda i: (0, 0)),
        ],
        out_specs=pl.BlockSpec((_BQ, D), lambda i: (i, 0)),
        out_shape=jax.ShapeDtypeStruct((N_Q, D), jnp.float32),
    )(z, scale_z, bias_z)


def kernel(ref_bxyz, query_bxyz, ref_feat, query_feat, e_ref, e_query,
           W_f0, gamma_f0, beta_f0, W_s0, gamma_s0, beta_s0,
           W1, b1, gamma1, beta1):
    feat2 = _ref_branch(ref_feat, W_f0, gamma_f0, beta_f0)
    ssum, ssq = _skip_stats(query_feat, W_s0)
    qf_acc, wsum = _sc_edge_conv(ref_bxyz, query_bxyz, e_ref, e_query, feat2)
    n = jnp.float32(N_Q)
    m_s = ssum / n
    v_s = ssq / n - m_s * m_s
    scale_s = gamma_s0.reshape(1, D) / jnp.sqrt(v_s + 1e-5)
    bias_s = beta_s0.reshape(1, D) - m_s * scale_s
    z, zsum, zsq = _mid(query_feat, W_s0, scale_s, bias_s, qf_acc, wsum, W1, b1)
    m_z = zsum / n
    v_z = zsq / n - m_z * m_z
    scale_z = gamma1.reshape(1, D) / jnp.sqrt(v_z + 1e-5)
    bias_z = beta1.reshape(1, D) - m_z * scale_z
    return _final(z, scale_z, bias_z)


# speedup vs baseline: 26.2560x; 1.0047x over previous
import functools

import jax
import jax.numpy as jnp
from jax import lax
from jax.experimental import pallas as pl
from jax.experimental.pallas import tpu as pltpu
from jax.experimental.pallas import tpu_sc as plsc

N_REF = 10000
N_Q = 100000
E = 300000
D = 128

NC, NS = 2, 16
NW = NC * NS
RW = 3136
NQ_PAD = NW * RW
WIN = 256
WSH = 8
KB = 128
EBLK = E // 16


def _rsqrt_nr(s):
    i = lax.bitcast_convert_type(s, jnp.int32)
    i = jnp.int32(0x5F3759DF) - (i >> 1)
    r = lax.bitcast_convert_type(i, jnp.float32)
    for _ in range(2):
        r = r * (1.5 - 0.5 * s * r * r)
    return r


def _sc_body(rx_h, ry_h, rz_h, qx_h, qy_h, qz_h, er_h, eq_h, eb_h, feat_h,
             zero_h, qf_h, ws_h,
             rx_v, ry_v, rz_v, qx_v, qy_v, qz_v,
             acc_v, wsw_v, feat_a, feat_b, er_a, eq_a, er_b, eq_b,
             d_a, qc_a, d_b, qc_b, eb_v,
             sem_ga, sem_gb, sem_f, sem_w, sem_b, sem_ia, sem_ib):
    wid = lax.axis_index("c") * NS + lax.axis_index("s")
    row_lo = wid * RW
    row_hi = jnp.minimum(row_lo + RW, N_Q)

    pltpu.async_copy(eb_h, eb_v, sem_ga)
    row_lo8 = pl.multiple_of(row_lo, 64)
    pltpu.async_copy(rx_h, rx_v, sem_b)
    pltpu.async_copy(ry_h, ry_v, sem_b)
    pltpu.async_copy(rz_h, rz_v, sem_b)
    pltpu.async_copy(qx_h.at[pl.ds(row_lo8, RW)], qx_v, sem_b)
    pltpu.async_copy(qy_h.at[pl.ds(row_lo8, RW)], qy_v, sem_b)
    pltpu.async_copy(qz_h.at[pl.ds(row_lo8, RW)], qz_v, sem_b)
    pltpu.async_copy(zero_h, acc_v, sem_b)
    pltpu.async_copy(zero_h.at[pl.ds(0, WIN)], wsw_v, sem_b)

    pltpu.make_async_copy(eb_h, eb_v, sem_ga).wait()
    widv = jnp.full((16,), wid, jnp.int32)
    e_lo = plsc.load_gather(eb_v, [widv])[0]
    e_hi = plsc.load_gather(eb_v, [widv + 1])[0]
    a_lo = e_lo & jnp.int32(-8)

    pltpu.make_async_copy(rx_h, rx_v, sem_b).wait()
    pltpu.make_async_copy(ry_h, ry_v, sem_b).wait()
    pltpu.make_async_copy(rz_h, rz_v, sem_b).wait()
    pltpu.make_async_copy(qx_h.at[pl.ds(row_lo8, RW)], qx_v, sem_b).wait()
    pltpu.make_async_copy(qy_h.at[pl.ds(row_lo8, RW)], qy_v, sem_b).wait()
    pltpu.make_async_copy(qz_h.at[pl.ds(row_lo8, RW)], qz_v, sem_b).wait()
    pltpu.make_async_copy(zero_h, acc_v, sem_b).wait()
    pltpu.make_async_copy(zero_h.at[pl.ds(0, WIN)], wsw_v, sem_b).wait()

    def flush(wb, nrows):
        n32 = nrows >> 5

        def issue(t, _):
            o1 = pl.multiple_of((wb + t * 32) * D, 4096)
            o2 = pl.multiple_of(wb + t * 32, 32)
            pltpu.async_copy(acc_v.at[pl.ds(t * 4096, 4096)],
                             qf_h.at[pl.ds(o1, 4096)], sem_f)
            pltpu.async_copy(wsw_v.at[pl.ds(t * 32, 32)],
                             ws_h.at[pl.ds(o2, 32)], sem_w)
            return 0

        def drain(t, _):
            o1 = pl.multiple_of((wb + t * 32) * D, 4096)
            o2 = pl.multiple_of(wb + t * 32, 32)
            pltpu.make_async_copy(acc_v.at[pl.ds(t * 4096, 4096)],
                                  qf_h.at[pl.ds(o1, 4096)],
                                  sem_f).wait()
            pltpu.make_async_copy(wsw_v.at[pl.ds(t * 32, 32)],
                                  ws_h.at[pl.ds(o2, 32)],
                                  sem_w).wait()
            pltpu.async_copy(zero_h.at[pl.ds(t * 4096, 4096)],
                             acc_v.at[pl.ds(t * 4096, 4096)], sem_b)
            pltpu.async_copy(zero_h.at[pl.ds(t * 32, 32)],
                             wsw_v.at[pl.ds(t * 32, 32)], sem_b)
            return 0

        def zwait(t, _):
            pltpu.make_async_copy(zero_h.at[pl.ds(t * 4096, 4096)],
                                  acc_v.at[pl.ds(t * 4096, 4096)],
                                  sem_b).wait()
            pltpu.make_async_copy(zero_h.at[pl.ds(t * 32, 32)],
                                  wsw_v.at[pl.ds(t * 32, 32)],
                                  sem_b).wait()
            return 0

        lax.fori_loop(0, n32, issue, 0)
        lax.fori_loop(0, n32, drain, 0)
        lax.fori_loop(0, n32, zwait, 0)

    def advance(wb):
        flush(wb, jnp.minimum(WIN, row_hi - wb))
        return wb + WIN

    lane = lax.iota(jnp.int32, 16)
    lane0 = lane == 0

    def groups(feat_v, d_v, qc_v, wb):
        def group_body(g, wb):
            eqb = qc_v[pl.ds(16 * g, 16)]
            db = d_v[pl.ds(16 * g, 16)]
            q0 = eqb[0]
            q15 = eqb[15]
            wb = lax.fori_loop(0, (q0 - wb) >> WSH,
                               lambda _, w: advance(w), wb)

            def fast(wb):
                for k in range(16):
                    ro = eqb[k] - wb
                    dv = jnp.full((16,), db[k])
                    for j in range(D // 16):
                        f = feat_v[16 * g + k, pl.ds(16 * j, 16)]
                        plsc.addupdate(acc_v.at[pl.ds(ro * D + 16 * j, 16)],
                                       f * dv)
                    plsc.addupdate_scatter(wsw_v,
                                           [jnp.full((16,), ro, jnp.int32)],
                                           db, mask=lane == k)
                return wb

            def slow(wb):
                def eb(k, w):
                    iv = jnp.full((16,), 16 * g + k, jnp.int32)
                    q = plsc.load_gather(qc_v, [iv])[0]
                    dbv = plsc.load_gather(d_v, [iv])
                    w = lax.fori_loop(0, (q - w) >> WSH,
                                      lambda _, x: advance(x), w)
                    ro = q - w
                    for j in range(D // 16):
                        f = feat_v[16 * g + k, pl.ds(16 * j, 16)]
                        plsc.addupdate(acc_v.at[pl.ds(ro * D + 16 * j, 16)],
                                       f * dbv)
                    plsc.addupdate_scatter(wsw_v,
                                           [jnp.full((16,), ro, jnp.int32)],
                                           dbv, mask=lane0)
                    return w

                return lax.fori_loop(0, 16, eb, wb)

            return lax.cond(q15 - wb < WIN, fast, slow, wb)

        return lax.fori_loop(0, KB // 16, group_body, wb)

    def issue_idx(b, er_v, eq_v, sem):
        base = pl.multiple_of(a_lo + b * KB, 8)
        pltpu.async_copy(er_h.at[pl.ds(base, KB)], er_v, sem)
        pltpu.async_copy(eq_h.at[pl.ds(base, KB)], eq_v, sem)

    def wait_idx(b, er_v, eq_v, sem):
        base = pl.multiple_of(a_lo + b * KB, 8)
        pltpu.make_async_copy(er_h.at[pl.ds(base, KB)], er_v, sem).wait()
        pltpu.make_async_copy(eq_h.at[pl.ds(base, KB)], eq_v, sem).wait()

    def weights(er_v, eq_v, d_v, qc_v):
        for g in range(KB // 16):
            qi = eq_v[pl.ds(16 * g, 16)]
            ri = er_v[pl.ds(16 * g, 16)]
            valid = (qi >= row_lo) & (qi < row_hi)
            qidx = jnp.where(valid, qi - row_lo, 0)
            qx = plsc.load_gather(qx_v, [qidx], mask=valid)
            qy = plsc.load_gather(qy_v, [qidx], mask=valid)
            qz = plsc.load_gather(qz_v, [qidx], mask=valid)
            rxx = plsc.load_gather(rx_v, [ri])
            ryy = plsc.load_gather(ry_v, [ri])
            rzz = plsc.load_gather(rz_v, [ri])
            dx = rxx - qx
            dy = ryy - qy
            dz = rzz - qz
            s = jnp.maximum(dx * dx + dy * dy + dz * dz, 1e-30)
            r = _rsqrt_nr(s)
            dd = r / (1.0 + 1e-8 * r)
            d_v[pl.ds(16 * g, 16)] = jnp.where(valid, dd, 0.0)
            qc_v[pl.ds(16 * g, 16)] = jnp.minimum(
                jnp.maximum(qi, row_lo), row_hi - 1)

    nblk = (e_hi - a_lo + (KB - 1)) >> 7
    npair = jnp.maximum((nblk + 1) >> 1, 1)
    issue_idx(0, er_a, eq_a, sem_ia)
    wait_idx(0, er_a, eq_a, sem_ia)
    pltpu.async_copy(feat_h.at[er_a], feat_a, sem_ga)
    issue_idx(1, er_b, eq_b, sem_ib)

    def pair_body(i, wb):
        wait_idx(2 * i + 1, er_b, eq_b, sem_ib)
        pltpu.async_copy(feat_h.at[er_b], feat_b, sem_gb)
        weights(er_a, eq_a, d_a, qc_a)
        pltpu.make_async_copy(feat_h.at[er_a], feat_a, sem_ga).wait()
        issue_idx(2 * i + 2, er_a, eq_a, sem_ia)
        wb = groups(feat_a, d_a, qc_a, wb)
        wait_idx(2 * i + 2, er_a, eq_a, sem_ia)
        pltpu.async_copy(feat_h.at[er_a], feat_a, sem_ga)
        weights(er_b, eq_b, d_b, qc_b)
        pltpu.make_async_copy(feat_h.at[er_b], feat_b, sem_gb).wait()
        issue_idx(2 * i + 3, er_b, eq_b, sem_ib)
        wb = groups(feat_b, d_b, qc_b, wb)
        return wb

    wb = lax.fori_loop(0, npair, pair_body, row_lo)
    pltpu.make_async_copy(feat_h.at[er_a], feat_a, sem_ga).wait()
    wait_idx(2 * npair + 1, er_b, eq_b, sem_ib)
    lax.fori_loop(0, (row_hi - wb + (WIN - 1)) >> WSH,
                  lambda _, w: advance(w), wb)


def _sc_edge_conv(ref_bxyz, query_bxyz, e_ref, e_query, feat):
    mesh = plsc.VectorSubcoreMesh(core_axis_name="c", subcore_axis_name="s")
    rx = ref_bxyz[:, 1] + 0.0
    ry = ref_bxyz[:, 2] + 0.0
    rz = ref_bxyz[:, 3] + 0.0
    qpad = jnp.pad(query_bxyz[:, 1:4], ((0, NQ_PAD - N_Q), (0, 0)))
    qx = qpad[:, 0] + 0.0
    qy = qpad[:, 1] + 0.0
    qz = qpad[:, 2] + 0.0
    er_p = jnp.pad(e_ref, (0, 6 * KB))
    eq_p = jnp.pad(e_query, (0, 6 * KB), constant_values=N_Q)
    bnd = jnp.minimum(jnp.arange(NW + 1, dtype=jnp.int32) * RW, N_Q)
    ebnd = jnp.searchsorted(e_query, bnd).astype(jnp.int32)
    ebnd = jnp.pad(ebnd, (0, 48 - (NW + 1)))
    zeros = jnp.zeros((WIN * D,), jnp.float32)
    run = pl.kernel(
        _sc_body,
        out_type=(jax.ShapeDtypeStruct((N_Q * D,), jnp.float32),
                  jax.ShapeDtypeStruct((N_Q,), jnp.float32)),
        mesh=mesh,
        compiler_params=pltpu.CompilerParams(needs_layout_passes=False),
        scratch_types=[
            pltpu.VMEM((N_REF,), jnp.float32),
            pltpu.VMEM((N_REF,), jnp.float32),
            pltpu.VMEM((N_REF,), jnp.float32),
            pltpu.VMEM((RW,), jnp.float32),
            pltpu.VMEM((RW,), jnp.float32),
            pltpu.VMEM((RW,), jnp.float32),
            pltpu.VMEM((WIN * D,), jnp.float32),
            pltpu.VMEM((WIN,), jnp.float32),
            pltpu.VMEM((KB, D), jnp.float32),
            pltpu.VMEM((KB, D), jnp.float32),
            pltpu.VMEM((KB,), jnp.int32),
            pltpu.VMEM((KB,), jnp.int32),
            pltpu.VMEM((KB,), jnp.int32),
            pltpu.VMEM((KB,), jnp.int32),
            pltpu.VMEM((KB,), jnp.float32),
            pltpu.VMEM((KB,), jnp.int32),
            pltpu.VMEM((KB,), jnp.float32),
            pltpu.VMEM((KB,), jnp.int32),
            pltpu.VMEM((48,), jnp.int32),
            pltpu.SemaphoreType.DMA,
            pltpu.SemaphoreType.DMA,
            pltpu.SemaphoreType.DMA,
            pltpu.SemaphoreType.DMA,
            pltpu.SemaphoreType.DMA,
            pltpu.SemaphoreType.DMA,
            pltpu.SemaphoreType.DMA,
        ],
    )
    qf_flat, wsum = run(rx, ry, rz, qx, qy, qz, er_p, eq_p, ebnd, feat, zeros)
    return qf_flat.reshape(N_Q, D), wsum


def _k1_body(x_ref, wt_ref, g_ref, b_ref, o_ref):
    y = jnp.dot(x_ref[...], wt_ref[...], preferred_element_type=jnp.float32)
    m = jnp.mean(y, axis=0, keepdims=True)
    v = jnp.mean((y - m) ** 2, axis=0, keepdims=True)
    o_ref[...] = g_ref[...] * (y - m) / jnp.sqrt(v + 1e-5) + b_ref[...]


def _ref_branch(ref_feat, W_f0, gamma, beta):
    return pl.pallas_call(
        _k1_body,
        out_shape=jax.ShapeDtypeStruct((N_REF, D), jnp.float32),
    )(ref_feat, W_f0.T, gamma.reshape(1, D), beta.reshape(1, D))


_BQ = 1000
_NBQ = N_Q // _BQ


def _stats_body(x_ref, wt_ref, ssum_ref, ssq_ref, acc_ref):
    i = pl.program_id(0)

    @pl.when(i == 0)
    def _():
        acc_ref[...] = jnp.zeros_like(acc_ref)

    y = jnp.dot(x_ref[...], wt_ref[...], preferred_element_type=jnp.float32)
    acc_ref[0:1, :] += jnp.sum(y, axis=0, keepdims=True)
    acc_ref[1:2, :] += jnp.sum(y * y, axis=0, keepdims=True)

    @pl.when(i == _NBQ - 1)
    def _():
        ssum_ref[...] = acc_ref[0:1, :]
        ssq_ref[...] = acc_ref[1:2, :]


def _skip_stats(query_feat, W_s0):
    return pl.pallas_call(
        _stats_body,
        grid=(_NBQ,),
        in_specs=[
            pl.BlockSpec((_BQ, D), lambda i: (i, 0)),
            pl.BlockSpec((D, D), lambda i: (0, 0)),
        ],
        out_specs=[
            pl.BlockSpec((1, D), lambda i: (0, 0)),
            pl.BlockSpec((1, D), lambda i: (0, 0)),
        ],
        out_shape=[
            jax.ShapeDtypeStruct((1, D), jnp.float32),
            jax.ShapeDtypeStruct((1, D), jnp.float32),
        ],
        scratch_shapes=[pltpu.VMEM((2, D), jnp.float32)],
    )(query_feat, W_s0.T)


def _mid_body(x_ref, wt_ref, sc_ref, bi_ref, acc_ref, w_ref, w1t_ref, b1_ref,
              z_ref, zsum_ref, zsq_ref, st_ref):
    i = pl.program_id(0)

    @pl.when(i == 0)
    def _():
        st_ref[...] = jnp.zeros_like(st_ref)

    ys = jnp.dot(x_ref[...], wt_ref[...], preferred_element_type=jnp.float32)
    skip = ys * sc_ref[...] + bi_ref[...]
    w = w_ref[...].reshape(_BQ, 1)
    inv = jnp.where(w > 0, 1.0 / w, 0.0)
    h = jnp.maximum(acc_ref[...] * inv + skip, 0.0)
    z = jnp.dot(h, w1t_ref[...], preferred_element_type=jnp.float32) + b1_ref[...]
    z_ref[...] = z
    st_ref[0:1, :] += jnp.sum(z, axis=0, keepdims=True)
    st_ref[1:2, :] += jnp.sum(z * z, axis=0, keepdims=True)

    @pl.when(i == _NBQ - 1)
    def _():
        zsum_ref[...] = st_ref[0:1, :]
        zsq_ref[...] = st_ref[1:2, :]


def _mid(query_feat, W_s0, scale_s, bias_s, qf_acc, wsum, W1, b1):
    return pl.pallas_call(
        _mid_body,
        grid=(_NBQ,),
        in_specs=[
            pl.BlockSpec((_BQ, D), lambda i: (i, 0)),
            pl.BlockSpec((D, D), lambda i: (0, 0)),
            pl.BlockSpec((1, D), lambda i: (0, 0)),
            pl.BlockSpec((1, D), lambda i: (0, 0)),
            pl.BlockSpec((_BQ, D), lambda i: (i, 0)),
            pl.BlockSpec((1, 1, _BQ), lambda i: (i, 0, 0)),
            pl.BlockSpec((D, D), lambda i: (0, 0)),
            pl.BlockSpec((1, D), lambda i: (0, 0)),
        ],
        out_specs=[
            pl.BlockSpec((_BQ, D), lambda i: (i, 0)),
            pl.BlockSpec((1, D), lambda i: (0, 0)),
            pl.BlockSpec((1, D), lambda i: (0, 0)),
        ],
        out_shape=[
            jax.ShapeDtypeStruct((N_Q, D), jnp.float32),
            jax.ShapeDtypeStruct((1, D), jnp.float32),
            jax.ShapeDtypeStruct((1, D), jnp.float32),
        ],
        scratch_shapes=[pltpu.VMEM((2, D), jnp.float32)],
    )(query_feat, W_s0.T, scale_s, bias_s, qf_acc,
      wsum.reshape(_NBQ, 1, _BQ), W1.T, b1.reshape(1, D))


def _fin_body(z_ref, sc_ref, bi_ref, o_ref):
    o_ref[...] = jnp.maximum(z_ref[...] * sc_ref[...] + bi_ref[...], 0.0)


def _final(z, scale_z, bias_z):
    return pl.pallas_call(
        _fin_body,
        grid=(_NBQ,),
        in_specs=[
            pl.BlockSpec((_BQ, D), lambda i: (i, 0)),
            pl.BlockSpec((1, D), lambda i: (0, 0)),
            pl.BlockSpec((1, D), lambda i: (0, 0)),
        ],
        out_specs=pl.BlockSpec((_BQ, D), lambda i: (i, 0)),
        out_shape=jax.ShapeDtypeStruct((N_Q, D), jnp.float32),
    )(z, scale_z, bias_z)


def kernel(ref_bxyz, query_bxyz, ref_feat, query_feat, e_ref, e_query,
           W_f0, gamma_f0, beta_f0, W_s0, gamma_s0, beta_s0,
           W1, b1, gamma1, beta1):
    feat2 = _ref_branch(ref_feat, W_f0, gamma_f0, beta_f0)
    ssum, ssq = _skip_stats(query_feat, W_s0)
    qf_acc, wsum = _sc_edge_conv(ref_bxyz, query_bxyz, e_ref, e_query, feat2)
    n = jnp.float32(N_Q)
    m_s = ssum / n
    v_s = ssq / n - m_s * m_s
    scale_s = gamma_s0.reshape(1, D) / jnp.sqrt(v_s + 1e-5)
    bias_s = beta_s0.reshape(1, D) - m_s * scale_s
    z, zsum, zsq = _mid(query_feat, W_s0, scale_s, bias_s, qf_acc, wsum, W1, b1)
    m_z = zsum / n
    v_z = zsq / n - m_z * m_z
    scale_z = gamma1.reshape(1, D) / jnp.sqrt(v_z + 1e-5)
    bias_z = beta1.reshape(1, D) - m_z * scale_z
    return _final(z, scale_z, bias_z)
